# Initial kernel scaffold; baseline (speedup 1.0000x reference)
import jax
import jax.numpy as jnp
from jax.experimental import pallas as pl


def kernel(x, edge_index, edge_attr, batch, params):
    raise NotImplementedError("write your pallas kernel here")


# speedup vs baseline: 5.6627x; 5.6627x over previous
import functools

import jax
import jax.numpy as jnp
from jax import lax
from jax.experimental import pallas as pl
from jax.experimental.pallas import tpu as pltpu
from jax.experimental.pallas import tpu_sc as plsc

_BN = 1000
_BE = 2000
_W = 125
_GB = 8
_NB = 16


def _iota16():
    return lax.broadcasted_iota(jnp.int32, (1, _NB), 1)


def _onehot_from_ids(ids):
    return (ids[:, None] == _iota16()).astype(jnp.float32)


def _onehot_from_src(src_f, counts_row):
    row = lax.broadcasted_iota(jnp.int32, (_NB, _NB), 0)
    col = lax.broadcasted_iota(jnp.int32, (_NB, _NB), 1)
    lt = (row < col).astype(jnp.float32)
    cum_excl = jnp.dot(counts_row, lt, preferred_element_type=jnp.float32, precision=lax.Precision.HIGHEST)
    upper = cum_excl + counts_row
    s = src_f[:, None]
    return ((s >= cum_excl) & (s < upper)).astype(jnp.float32)


def _prep1_body(x_ref, ws_ref, wd_ref, b1_ref, batch_ref, xs_ref, xd_ref, cnt_ref):
    x = x_ref[...]
    xs_ref[...] = jnp.dot(x, ws_ref[...], preferred_element_type=jnp.float32) + b1_ref[...]
    xd_ref[...] = jnp.dot(x, wd_ref[...], preferred_element_type=jnp.float32)
    oh = _onehot_from_ids(batch_ref[0, 0, :])
    cnt = jnp.sum(oh, axis=0)

    @pl.when(pl.program_id(0) == 0)
    def _():
        cnt_ref[...] = jnp.zeros_like(cnt_ref)

    cnt_ref[...] += jnp.concatenate(
        [cnt[None, :], jnp.zeros((7, _NB), jnp.float32)], axis=0)


def _prep1(x, ws, wd, b1, batch_r, interpret=False):
    n, d = x.shape
    g = n // _BN
    return pl.pallas_call(
        _prep1_body,
        grid=(g,),
        in_specs=[
            pl.BlockSpec((_BN, d), lambda i: (i, 0)),
            pl.BlockSpec((d, 64), lambda i: (0, 0)),
            pl.BlockSpec((d, 64), lambda i: (0, 0)),
            pl.BlockSpec((1, 64), lambda i: (0, 0)),
            pl.BlockSpec((1, 1, _BN), lambda i: (i, 0, 0)),
        ],
        out_specs=[
            pl.BlockSpec((_BN, 64), lambda i: (i, 0)),
            pl.BlockSpec((_BN, 64), lambda i: (i, 0)),
            pl.BlockSpec((8, _NB), lambda i: (0, 0)),
        ],
        out_shape=[
            jax.ShapeDtypeStruct((n, 64), jnp.float32),
            jax.ShapeDtypeStruct((n, 64), jnp.float32),
            jax.ShapeDtypeStruct((8, _NB), jnp.float32),
        ],
        interpret=interpret,
    )(x, ws, wd, b1, batch_r)


def _prep2_body(x_ref, ws_ref, wd_ref, b1_ref, xs_ref, xd_ref):
    x = x_ref[...]
    xs_ref[...] = jnp.dot(x, ws_ref[...], preferred_element_type=jnp.float32) + b1_ref[...]
    xd_ref[...] = jnp.dot(x, wd_ref[...], preferred_element_type=jnp.float32)


def _prep2(x, ws, wd, b1, interpret=False):
    n, d = x.shape
    g = n // _BN
    return pl.pallas_call(
        _prep2_body,
        grid=(g,),
        in_specs=[
            pl.BlockSpec((_BN, d), lambda i: (i, 0)),
            pl.BlockSpec((d, 64), lambda i: (0, 0)),
            pl.BlockSpec((d, 64), lambda i: (0, 0)),
            pl.BlockSpec((1, 64), lambda i: (0, 0)),
        ],
        out_specs=[
            pl.BlockSpec((_BN, 64), lambda i: (i, 0)),
            pl.BlockSpec((_BN, 64), lambda i: (i, 0)),
        ],
        out_shape=[
            jax.ShapeDtypeStruct((n, 64), jnp.float32),
            jax.ShapeDtypeStruct((n, 64), jnp.float32),
        ],
        interpret=interpret,
    )(x, ws, wd, b1)


def _edge1_body(xsg_ref, xdg_ref, ea_ref, src_ref, cnt_ref, w1e_ref, w2_ref,
                b2_ref, eo_ref, ge_ref):
    oh = _onehot_from_src(src_ref[0, 0, :].astype(jnp.float32), cnt_ref[0:1, :])
    h = xsg_ref[...] + xdg_ref[...] + jnp.dot(
        ea_ref[...], w1e_ref[...], preferred_element_type=jnp.float32)
    h = jnp.maximum(h, 0.0)
    ea = jnp.dot(h, w2_ref[...], preferred_element_type=jnp.float32) + b2_ref[...]
    eo_ref[...] = ea
    gs = lax.dot_general(oh, ea, (((0,), (0,)), ((), ())),
                         preferred_element_type=jnp.float32,
                         precision=lax.Precision.HIGHEST)
    ecnt = jnp.sum(oh, axis=0)
    upd = jnp.concatenate(
        [gs, ecnt[None, :], jnp.zeros((7, _NB), jnp.float32)], axis=0)

    @pl.when(pl.program_id(0) == 0)
    def _():
        ge_ref[...] = jnp.zeros_like(ge_ref)

    ge_ref[...] += upd


def _edge1(xsg, xdg, ea_in, src_r, cnt, w1e, w2, b2, interpret=False):
    e = xsg.shape[0]
    ed = ea_in.shape[1]
    g = e // _BE
    return pl.pallas_call(
        _edge1_body,
        grid=(g,),
        in_specs=[
            pl.BlockSpec((_BE, 64), lambda i: (i, 0)),
            pl.BlockSpec((_BE, 64), lambda i: (i, 0)),
            pl.BlockSpec((_BE, ed), lambda i: (i, 0)),
            pl.BlockSpec((1, 1, _BE), lambda i: (i, 0, 0)),
            pl.BlockSpec((8, _NB), lambda i: (0, 0)),
            pl.BlockSpec((ed, 64), lambda i: (0, 0)),
            pl.BlockSpec((64, 16), lambda i: (0, 0)),
            pl.BlockSpec((1, 16), lambda i: (0, 0)),
        ],
        out_specs=[
            pl.BlockSpec((_BE, 16), lambda i: (i, 0)),
            pl.BlockSpec((24, _NB), lambda i: (0, 0)),
        ],
        out_shape=[
            jax.ShapeDtypeStruct((e, 16), jnp.float32),
            jax.ShapeDtypeStruct((24, _NB), jnp.float32),
        ],
        interpret=interpret,
    )(xsg, xdg, ea_in, src_r, cnt, w1e, w2, b2)


def _edge2_body(xsg_ref, xdg_ref, ea_ref, src_ref, cnt_ref, w1e_ref, uw_ref,
                w2_ref, b2_ref, eo_ref):
    oh = _onehot_from_src(src_ref[0, 0, :].astype(jnp.float32), cnt_ref[0:1, :])
    h = xsg_ref[...] + xdg_ref[...] + jnp.dot(
        ea_ref[...], w1e_ref[...], preferred_element_type=jnp.float32)
    h += jnp.dot(oh, uw_ref[...], preferred_element_type=jnp.float32,
                 precision=lax.Precision.HIGHEST)
    h = jnp.maximum(h, 0.0)
    eo_ref[...] = jnp.dot(h, w2_ref[...], preferred_element_type=jnp.float32) + b2_ref[...]


def _edge2(xsg, xdg, ea_in, src_r, cnt, w1e, uw, w2, b2, interpret=False):
    e = xsg.shape[0]
    ed = ea_in.shape[1]
    g = e // _BE
    return pl.pallas_call(
        _edge2_body,
        grid=(g,),
        in_specs=[
            pl.BlockSpec((_BE, 64), lambda i: (i, 0)),
            pl.BlockSpec((_BE, 64), lambda i: (i, 0)),
            pl.BlockSpec((_BE, ed), lambda i: (i, 0)),
            pl.BlockSpec((1, 1, _BE), lambda i: (i, 0, 0)),
            pl.BlockSpec((8, _NB), lambda i: (0, 0)),
            pl.BlockSpec((ed, 64), lambda i: (0, 0)),
            pl.BlockSpec((_NB, 64), lambda i: (0, 0)),
            pl.BlockSpec((64, 16), lambda i: (0, 0)),
            pl.BlockSpec((1, 16), lambda i: (0, 0)),
        ],
        out_specs=[pl.BlockSpec((_BE, 16), lambda i: (i, 0))],
        out_shape=[jax.ShapeDtypeStruct((e, 16), jnp.float32)],
        interpret=interpret,
    )(xsg, xdg, ea_in, src_r, cnt, w1e, uw, w2, b2)


def _node1_body(x_ref, p0_ref, p1_ref, c0_ref, c1_ref, batch_ref, w1x_ref,
                w1a_ref, b1_ref, w2_ref, b2_ref, xo_ref, gx_ref):
    deg = (c0_ref[...] + c1_ref[...])[:, 0:1]
    agg = (p0_ref[...] + p1_ref[...]) / jnp.maximum(deg, 1.0)
    oh = _onehot_from_ids(batch_ref[0, 0, :])
    h = (jnp.dot(x_ref[...], w1x_ref[...], preferred_element_type=jnp.float32)
         + jnp.dot(agg, w1a_ref[...], preferred_element_type=jnp.float32)
         + b1_ref[...])
    h = jnp.maximum(h, 0.0)
    xo = jnp.dot(h, w2_ref[...], preferred_element_type=jnp.float32) + b2_ref[...]
    xo_ref[...] = xo
    gs = lax.dot_general(oh, xo, (((0,), (0,)), ((), ())),
                         preferred_element_type=jnp.float32,
                         precision=lax.Precision.HIGHEST)

    @pl.when(pl.program_id(0) == 0)
    def _():
        gx_ref[...] = jnp.zeros_like(gx_ref)

    gx_ref[...] += gs


def _node1(x, p0, p1, c0, c1, batch_r, w1x, w1a, b1, w2, b2, interpret=False):
    n, d = x.shape
    g = n // _BN
    return pl.pallas_call(
        _node1_body,
        grid=(g,),
        in_specs=[
            pl.BlockSpec((_BN, d), lambda i: (i, 0)),
            pl.BlockSpec((_BN, 16), lambda i: (i, 0)),
            pl.BlockSpec((_BN, 16), lambda i: (i, 0)),
            pl.BlockSpec((_BN, 16), lambda i: (i, 0)),
            pl.BlockSpec((_BN, 16), lambda i: (i, 0)),
            pl.BlockSpec((1, 1, _BN), lambda i: (i, 0, 0)),
            pl.BlockSpec((d, 64), lambda i: (0, 0)),
            pl.BlockSpec((16, 64), lambda i: (0, 0)),
            pl.BlockSpec((1, 64), lambda i: (0, 0)),
            pl.BlockSpec((64, d), lambda i: (0, 0)),
            pl.BlockSpec((1, d), lambda i: (0, 0)),
        ],
        out_specs=[
            pl.BlockSpec((_BN, d), lambda i: (i, 0)),
            pl.BlockSpec((_NB, d), lambda i: (0, 0)),
        ],
        out_shape=[
            jax.ShapeDtypeStruct((n, d), jnp.float32),
            jax.ShapeDtypeStruct((_NB, d), jnp.float32),
        ],
        interpret=interpret,
    )(x, p0, p1, c0, c1, batch_r, w1x, w1a, b1, w2, b2)


def _node2_body(x_ref, p0_ref, p1_ref, c0_ref, c1_ref, batch_ref, w1x_ref,
                w1a_ref, uwn_ref, b1_ref, w2_ref, b2_ref, gx_ref):
    deg = (c0_ref[...] + c1_ref[...])[:, 0:1]
    agg = (p0_ref[...] + p1_ref[...]) / jnp.maximum(deg, 1.0)
    oh = _onehot_from_ids(batch_ref[0, 0, :])
    h = (jnp.dot(x_ref[...], w1x_ref[...], preferred_element_type=jnp.float32)
         + jnp.dot(agg, w1a_ref[...], preferred_element_type=jnp.float32)
         + jnp.dot(oh, uwn_ref[...], preferred_element_type=jnp.float32,
                   precision=lax.Precision.HIGHEST)
         + b1_ref[...])
    h = jnp.maximum(h, 0.0)
    xo = jnp.dot(h, w2_ref[...], preferred_element_type=jnp.float32) + b2_ref[...]
    gs = lax.dot_general(oh, xo, (((0,), (0,)), ((), ())),
                         preferred_element_type=jnp.float32,
                         precision=lax.Precision.HIGHEST)

    @pl.when(pl.program_id(0) == 0)
    def _():
        gx_ref[...] = jnp.zeros_like(gx_ref)

    gx_ref[...] += gs


def _node2(x, p0, p1, c0, c1, batch_r, w1x, w1a, uwn, b1, w2, b2, interpret=False):
    n, d = x.shape
    g = n // _BN
    return pl.pallas_call(
        _node2_body,
        grid=(g,),
        in_specs=[
            pl.BlockSpec((_BN, d), lambda i: (i, 0)),
            pl.BlockSpec((_BN, 16), lambda i: (i, 0)),
            pl.BlockSpec((_BN, 16), lambda i: (i, 0)),
            pl.BlockSpec((_BN, 16), lambda i: (i, 0)),
            pl.BlockSpec((_BN, 16), lambda i: (i, 0)),
            pl.BlockSpec((1, 1, _BN), lambda i: (i, 0, 0)),
            pl.BlockSpec((d, 64), lambda i: (0, 0)),
            pl.BlockSpec((16, 64), lambda i: (0, 0)),
            pl.BlockSpec((_NB, 64), lambda i: (0, 0)),
            pl.BlockSpec((1, 64), lambda i: (0, 0)),
            pl.BlockSpec((64, d), lambda i: (0, 0)),
            pl.BlockSpec((1, d), lambda i: (0, 0)),
        ],
        out_specs=[pl.BlockSpec((_NB, d), lambda i: (0, 0))],
        out_shape=[jax.ShapeDtypeStruct((_NB, d), jnp.float32)],
        interpret=interpret,
    )(x, p0, p1, c0, c1, batch_r, w1x, w1a, uwn, b1, w2, b2)


def _glob1_body(cnt_ref, ge_ref, gx_ref, wg_gx_ref, wg_ge_ref, b1_ref, w2_ref,
                b2_ref, we_ref, wn_ref, uwe_ref, uwn_ref):
    ncnt = cnt_ref[0:1, :]
    gx = gx_ref[...] / jnp.maximum(ncnt, 1.0).reshape(_NB, 1)
    ecnt = ge_ref[16:17, :]
    ge = ge_ref[0:16, :] / jnp.maximum(ecnt, 1.0).reshape(_NB, 1)
    h = (jnp.dot(gx, wg_gx_ref[...], preferred_element_type=jnp.float32)
         + jnp.dot(ge, wg_ge_ref[...], preferred_element_type=jnp.float32)
         + b1_ref[...])
    h = jnp.maximum(h, 0.0)
    u1 = jnp.dot(h, w2_ref[...], preferred_element_type=jnp.float32) + b2_ref[...]
    uwe_ref[...] = jnp.dot(u1, we_ref[...], preferred_element_type=jnp.float32)
    uwn_ref[...] = jnp.dot(u1, wn_ref[...], preferred_element_type=jnp.float32)


def _glob1(cnt, ge, gx, wg_gx, wg_ge, b1, w2, b2, we, wn, interpret=False):
    d = gx.shape[1]
    return pl.pallas_call(
        _glob1_body,
        out_shape=[
            jax.ShapeDtypeStruct((_NB, 64), jnp.float32),
            jax.ShapeDtypeStruct((_NB, 64), jnp.float32),
        ],
        interpret=interpret,
    )(cnt, ge, gx, wg_gx, wg_ge, b1, w2, b2, we, wn)


def _bn16(h, g, b):
    m = jnp.mean(h, axis=0, keepdims=True)
    v = jnp.mean((h - m) ** 2, axis=0, keepdims=True)
    return g * (h - m) / jnp.sqrt(v + 1e-5) + b


def _head_body(gx_ref, cnt_ref, aw1_ref, ab1_ref, ag1_ref, abe1_ref, aw2_ref,
               ab2_ref, ow1_ref, ob1_ref, og1_ref, obe1_ref, ow2_ref, ob2_ref,
               og2_ref, obe2_ref, ow3_ref, ob3_ref, act_ref, obj_ref):
    maxn = jnp.max(cnt_ref[0:1, :])
    outputs = gx_ref[...] / maxn
    h = _bn16(jnp.dot(outputs, aw1_ref[...], preferred_element_type=jnp.float32)
              + ab1_ref[...], ag1_ref[...], abe1_ref[...])
    act_ref[...] = jnp.dot(jnp.maximum(h, 0.0), aw2_ref[...],
                           preferred_element_type=jnp.float32) + ab2_ref[...]
    h = jnp.maximum(_bn16(
        jnp.dot(outputs, ow1_ref[...], preferred_element_type=jnp.float32)
        + ob1_ref[...], og1_ref[...], obe1_ref[...]), 0.0)
    h = jnp.maximum(_bn16(
        jnp.dot(h, ow2_ref[...], preferred_element_type=jnp.float32)
        + ob2_ref[...], og2_ref[...], obe2_ref[...]), 0.0)
    obj_ref[...] = jnp.dot(h, ow3_ref[...],
                           preferred_element_type=jnp.float32) + ob3_ref[...]


def _head(gx2, cnt, pa, po, interpret=False):
    args = (gx2, cnt,
            pa["W1"], pa["b1"].reshape(1, -1), pa["g1"].reshape(1, -1),
            pa["be1"].reshape(1, -1), pa["W2"], pa["b2"].reshape(1, -1),
            po["W1"], po["b1"].reshape(1, -1), po["g1"].reshape(1, -1),
            po["be1"].reshape(1, -1), po["W2"], po["b2"].reshape(1, -1),
            po["g2"].reshape(1, -1), po["be2"].reshape(1, -1), po["W3"],
            po["b3"].reshape(1, -1))
    return pl.pallas_call(
        _head_body,
        out_shape=[
            jax.ShapeDtypeStruct((_NB, 32), jnp.float32),
            jax.ShapeDtypeStruct((_NB, 64), jnp.float32),
        ],
        interpret=interpret,
    )(*args)


@functools.lru_cache(maxsize=None)
def _build_sc_gather(n, e):
    mesh = plsc.VectorSubcoreMesh(core_axis_name="c", subcore_axis_name="s")
    epw = e // 32
    nrow = epw // _W
    ng = nrow // _GB
    grp = _GB * _W

    @functools.partial(
        pl.kernel, mesh=mesh,
        compiler_params=pltpu.CompilerParams(use_tc_tiling_on_sc=False),
        out_type=(jax.ShapeDtypeStruct((e, 64), jnp.float32),
                  jax.ShapeDtypeStruct((e, 64), jnp.float32)),
        scratch_types=[
            pltpu.VMEM((nrow, _W), jnp.int32),
            pltpu.VMEM((nrow, _W), jnp.int32),
            pltpu.VMEM((grp, 64), jnp.float32),
            pltpu.SemaphoreType.DMA,
        ],
    )
    def gk(xs_hbm, xd_hbm, src_hbm, dst_hbm, xsg_hbm, xdg_hbm,
           idxs, idxd, rows, sem):
        wid = lax.axis_index("s") * 2 + lax.axis_index("c")
        tb = wid * nrow
        base = wid * epw
        pltpu.sync_copy(src_hbm.at[pl.ds(tb, nrow)], idxs)
        pltpu.sync_copy(dst_hbm.at[pl.ds(tb, nrow)], idxd)

        def group(g, carry):
            off = base + g * grp
            cps = [pltpu.async_copy(xs_hbm.at[idxs.at[g * _GB + b]],
                                    rows.at[pl.ds(b * _W, _W)], sem)
                   for b in range(_GB)]
            for cp in cps:
                cp.wait()
            pltpu.sync_copy(rows, xsg_hbm.at[pl.ds(off, grp)])
            cps = [pltpu.async_copy(xd_hbm.at[idxd.at[g * _GB + b]],
                                    rows.at[pl.ds(b * _W, _W)], sem)
                   for b in range(_GB)]
            for cp in cps:
                cp.wait()
            pltpu.sync_copy(rows, xdg_hbm.at[pl.ds(off, grp)])
            return carry

        lax.fori_loop(0, ng, group, 0)

    return gk


@functools.lru_cache(maxsize=None)
def _build_sc_scatter(n, e, with_cnt):
    mesh = plsc.VectorSubcoreMesh(core_axis_name="c", subcore_axis_name="s")
    epw = e // 32
    nrow = epw // _W
    ng = nrow // _GB
    grp = _GB * _W
    nzw = 10
    rps = n // nzw

    outs = [jax.ShapeDtypeStruct((2, n, 16), jnp.float32)]
    scratch = [
        pltpu.VMEM((nrow, _W), jnp.int32),
        pltpu.VMEM((grp, 16), jnp.float32),
        pltpu.VMEM_SHARED((n, 16), jnp.float32),
    ]
    if with_cnt:
        outs.append(jax.ShapeDtypeStruct((2, n, 16), jnp.float32))
        scratch += [
            pltpu.VMEM((_W, 16), jnp.float32),
            pltpu.VMEM_SHARED((n, 16), jnp.float32),
        ]

    def body(ea_hbm, dst_hbm, zeros_hbm, ones_hbm, agg_hbm, cnt_hbm,
             idxd, rows, accum, obuf, caccum):
        cid = lax.axis_index("c")
        sid = lax.axis_index("s")
        wid = sid * 2 + cid

        @pl.when(sid < nzw)
        def _():
            pltpu.sync_copy(zeros_hbm, accum.at[pl.ds(sid * rps, rps)])
            if with_cnt:
                pltpu.sync_copy(zeros_hbm, caccum.at[pl.ds(sid * rps, rps)])

        if with_cnt:
            pltpu.sync_copy(ones_hbm, obuf)
        plsc.subcore_barrier()
        tb = wid * nrow
        base = wid * epw
        pltpu.sync_copy(dst_hbm.at[pl.ds(tb, nrow)], idxd)

        def group(g, carry):
            pltpu.sync_copy(ea_hbm.at[pl.ds(base + g * grp, grp)], rows)
            for b in range(_GB):
                pltpu.sync_copy(rows.at[pl.ds(b * _W, _W)],
                                accum.at[idxd.at[g * _GB + b]], add=True)
                if with_cnt:
                    pltpu.sync_copy(obuf, caccum.at[idxd.at[g * _GB + b]],
                                    add=True)
            return carry

        lax.fori_loop(0, ng, group, 0)
        plsc.subcore_barrier()

        @pl.when(sid < nzw)
        def _():
            pltpu.sync_copy(accum.at[pl.ds(sid * rps, rps)],
                            agg_hbm.at[cid, pl.ds(sid * rps, rps)])
            if with_cnt:
                pltpu.sync_copy(caccum.at[pl.ds(sid * rps, rps)],
                                cnt_hbm.at[cid, pl.ds(sid * rps, rps)])

    if with_cnt:
        def sk(ea_hbm, dst_hbm, zeros_hbm, ones_hbm, agg_hbm, cnt_hbm,
               idxd, rows, accum, obuf, caccum):
            body(ea_hbm, dst_hbm, zeros_hbm, ones_hbm, agg_hbm, cnt_hbm,
                 idxd, rows, accum, obuf, caccum)
    else:
        def sk(ea_hbm, dst_hbm, zeros_hbm, agg_hbm, idxd, rows, accum):
            body(ea_hbm, dst_hbm, zeros_hbm, None, agg_hbm, None,
                 idxd, rows, accum, None, None)

    return functools.partial(
        pl.kernel, mesh=mesh, out_type=tuple(outs),
        compiler_params=pltpu.CompilerParams(use_tc_tiling_on_sc=False),
        scratch_types=scratch)(sk)


def kernel(x, edge_index, edge_attr, batch, params):
    n, d = x.shape
    e = edge_index.shape[1]
    src = edge_index[0].astype(jnp.int32)
    dst = edge_index[1].astype(jnp.int32)
    src2d = src.reshape(e // _W, _W)
    dst2d = dst.reshape(e // _W, _W)
    src_r = src.reshape(e // _BE, 1, _BE)
    batch_r = batch.astype(jnp.int32).reshape(n // _BN, 1, _BN)
    zeros_np = jnp.zeros((n // 10, 16), jnp.float32)
    ones_w = jnp.ones((_W, 16), jnp.float32)

    p1, p2 = params["gnn1"], params["gnn2"]
    pe1, pn1, pg1 = p1["edge"], p1["node"], p1["glob"]
    pe2, pn2 = p2["edge"], p2["node"]

    gather = _build_sc_gather(n, e)
    scatter1 = _build_sc_scatter(n, e, True)
    scatter2 = _build_sc_scatter(n, e, False)

    xs1, xd1, cnt = _prep1(x, pe1["W1"][:d], pe1["W1"][d:2 * d],
                           pe1["b1"].reshape(1, -1), batch_r)
    xsg1, xdg1 = gather(xs1, xd1, src2d, dst2d)
    ea1, ge = _edge1(xsg1, xdg1, edge_attr, src_r, cnt,
                     pe1["W1"][2 * d:2 * d + 16], pe1["W2"],
                     pe1["b2"].reshape(1, -1))
    aggp, cntp = scatter1(ea1, dst2d, zeros_np, ones_w)
    x1, gx1 = _node1(x, aggp[0], aggp[1], cntp[0], cntp[1], batch_r,
                     pn1["W1"][:d], pn1["W1"][d:d + 16],
                     pn1["b1"].reshape(1, -1), pn1["W2"],
                     pn1["b2"].reshape(1, -1))
    uwe2, uwn2 = _glob1(cnt, ge, gx1, pg1["W1"][16:16 + d],
                        pg1["W1"][16 + d:], pg1["b1"].reshape(1, -1),
                        pg1["W2"], pg1["b2"].reshape(1, -1),
                        pe2["W1"][2 * d + 16:], pn2["W1"][d + 16:])

    xs2, xd2 = _prep2(x1, pe2["W1"][:d], pe2["W1"][d:2 * d],
                      pe2["b1"].reshape(1, -1))
    xsg2, xdg2 = gather(xs2, xd2, src2d, dst2d)
    (ea2,) = _edge2(xsg2, xdg2, ea1, src_r, cnt,
                    pe2["W1"][2 * d:2 * d + 16], uwe2, pe2["W2"],
                    pe2["b2"].reshape(1, -1))
    (aggp2,) = scatter2(ea2, dst2d, zeros_np)
    (gx2,) = _node2(x1, aggp2[0], aggp2[1], cntp[0], cntp[1], batch_r,
                    pn2["W1"][:d], pn2["W1"][d:d + 16], uwn2,
                    pn2["b1"].reshape(1, -1), pn2["W2"],
                    pn2["b2"].reshape(1, -1))
    act, obj = _head(gx2, cnt, params["action"], params["object"])
    return act, obj


# speedup vs baseline: 6.0117x; 1.0616x over previous
import functools

import jax
import jax.numpy as jnp
from jax import lax
from jax.experimental import pallas as pl
from jax.experimental.pallas import tpu as pltpu
from jax.experimental.pallas import tpu_sc as plsc

_BN = 1000
_BE = 2000
_W = 125
_GB = 8
_NB = 16
_NW = 32

_HI = lax.Precision.HIGHEST


def _iota16():
    return lax.broadcasted_iota(jnp.int32, (1, _NB), 1)


def _onehot_from_ids(ids):
    return (ids[:, None] == _iota16()).astype(jnp.float32)


def _onehot_t_from_src(src_f, counts_row):
    row = lax.broadcasted_iota(jnp.int32, (_NB, _NB), 0)
    col = lax.broadcasted_iota(jnp.int32, (_NB, _NB), 1)
    lt = (row < col).astype(jnp.float32)
    cum_excl = jnp.dot(counts_row, lt, preferred_element_type=jnp.float32,
                       precision=_HI)
    upper = cum_excl + counts_row
    s = src_f[None, :]
    return ((s >= cum_excl.reshape(_NB, 1)) & (s < upper.reshape(_NB, 1))
            ).astype(jnp.float32)


def _prep1_body(x_ref, ws_ref, wd_ref, b1_ref, batch_ref, xs_ref, xd_ref, cnt_ref):
    x = x_ref[...]
    xs_ref[...] = jnp.dot(x, ws_ref[...], preferred_element_type=jnp.float32) + b1_ref[...]
    xd_ref[...] = jnp.dot(x, wd_ref[...], preferred_element_type=jnp.float32)
    oh = _onehot_from_ids(batch_ref[0, 0, :])
    cnt = jnp.sum(oh, axis=0)

    @pl.when(pl.program_id(0) == 0)
    def _():
        cnt_ref[...] = jnp.zeros_like(cnt_ref)

    cnt_ref[...] += jnp.concatenate(
        [cnt[None, :], jnp.zeros((7, _NB), jnp.float32)], axis=0)


def _prep1(x, ws, wd, b1, batch_r, interpret=False):
    n, d = x.shape
    g = n // _BN
    return pl.pallas_call(
        _prep1_body,
        grid=(g,),
        in_specs=[
            pl.BlockSpec((_BN, d), lambda i: (i, 0)),
            pl.BlockSpec((d, 64), lambda i: (0, 0)),
            pl.BlockSpec((d, 64), lambda i: (0, 0)),
            pl.BlockSpec((1, 64), lambda i: (0, 0)),
            pl.BlockSpec((1, 1, _BN), lambda i: (i, 0, 0)),
        ],
        out_specs=[
            pl.BlockSpec((_BN, 64), lambda i: (i, 0)),
            pl.BlockSpec((_BN, 64), lambda i: (i, 0)),
            pl.BlockSpec((8, _NB), lambda i: (0, 0)),
        ],
        out_shape=[
            jax.ShapeDtypeStruct((n, 64), jnp.float32),
            jax.ShapeDtypeStruct((n, 64), jnp.float32),
            jax.ShapeDtypeStruct((8, _NB), jnp.float32),
        ],
        interpret=interpret,
    )(x, ws, wd, b1, batch_r)


def _prep2_body(x_ref, ws_ref, wd_ref, b1_ref, batch_ref, uwe_ref, xs_ref, xd_ref):
    x = x_ref[...]
    oh = _onehot_from_ids(batch_ref[0, 0, :])
    xs_ref[...] = (jnp.dot(x, ws_ref[...], preferred_element_type=jnp.float32)
                   + jnp.dot(oh, uwe_ref[...], preferred_element_type=jnp.float32,
                             precision=_HI)
                   + b1_ref[...])
    xd_ref[...] = jnp.dot(x, wd_ref[...], preferred_element_type=jnp.float32)


def _prep2(x, ws, wd, b1, batch_r, uwe, interpret=False):
    n, d = x.shape
    g = n // _BN
    return pl.pallas_call(
        _prep2_body,
        grid=(g,),
        in_specs=[
            pl.BlockSpec((_BN, d), lambda i: (i, 0)),
            pl.BlockSpec((d, 64), lambda i: (0, 0)),
            pl.BlockSpec((d, 64), lambda i: (0, 0)),
            pl.BlockSpec((1, 64), lambda i: (0, 0)),
            pl.BlockSpec((1, 1, _BN), lambda i: (i, 0, 0)),
            pl.BlockSpec((_NB, 64), lambda i: (0, 0)),
        ],
        out_specs=[
            pl.BlockSpec((_BN, 64), lambda i: (i, 0)),
            pl.BlockSpec((_BN, 64), lambda i: (i, 0)),
        ],
        out_shape=[
            jax.ShapeDtypeStruct((n, 64), jnp.float32),
            jax.ShapeDtypeStruct((n, 64), jnp.float32),
        ],
        interpret=interpret,
    )(x, ws, wd, b1, batch_r, uwe)


def _edge1_body(xsg_ref, xdg_ref, ea_ref, src_ref, cnt_ref, w1e_ref, w2_ref,
                b2_ref, eo_ref, ge_ref):
    oht = _onehot_t_from_src(src_ref[0, 0, :].astype(jnp.float32), cnt_ref[0:1, :])
    h = xsg_ref[...] + xdg_ref[...] + jnp.dot(
        ea_ref[...], w1e_ref[...], preferred_element_type=jnp.float32)
    h = jnp.maximum(h, 0.0)
    ea = jnp.dot(h, w2_ref[...], preferred_element_type=jnp.float32) + b2_ref[...]
    eo_ref[...] = ea
    gs = jnp.dot(oht, ea, preferred_element_type=jnp.float32, precision=_HI)
    ecnt = jnp.sum(oht, axis=1)
    upd = jnp.concatenate(
        [gs, ecnt[None, :], jnp.zeros((7, _NB), jnp.float32)], axis=0)

    @pl.when(pl.program_id(0) == 0)
    def _():
        ge_ref[...] = jnp.zeros_like(ge_ref)

    ge_ref[...] += upd


def _edge1(xsg, xdg, ea_in, src_r, cnt, w1e, w2, b2, interpret=False):
    e = xsg.shape[0]
    ed = ea_in.shape[1]
    g = e // _BE
    return pl.pallas_call(
        _edge1_body,
        grid=(g,),
        in_specs=[
            pl.BlockSpec((_BE, 64), lambda i: (i, 0)),
            pl.BlockSpec((_BE, 64), lambda i: (i, 0)),
            pl.BlockSpec((_BE, ed), lambda i: (i, 0)),
            pl.BlockSpec((1, 1, _BE), lambda i: (i, 0, 0)),
            pl.BlockSpec((8, _NB), lambda i: (0, 0)),
            pl.BlockSpec((ed, 64), lambda i: (0, 0)),
            pl.BlockSpec((64, 16), lambda i: (0, 0)),
            pl.BlockSpec((1, 16), lambda i: (0, 0)),
        ],
        out_specs=[
            pl.BlockSpec((_BE, 16), lambda i: (i, 0)),
            pl.BlockSpec((24, _NB), lambda i: (0, 0)),
        ],
        out_shape=[
            jax.ShapeDtypeStruct((e, 16), jnp.float32),
            jax.ShapeDtypeStruct((24, _NB), jnp.float32),
        ],
        interpret=interpret,
    )(xsg, xdg, ea_in, src_r, cnt, w1e, w2, b2)


def _edge2_body(xsg_ref, xdg_ref, ea_ref, w1e_ref, w2_ref, b2_ref, eo_ref):
    h = xsg_ref[...] + xdg_ref[...] + jnp.dot(
        ea_ref[...], w1e_ref[...], preferred_element_type=jnp.float32)
    h = jnp.maximum(h, 0.0)
    eo_ref[...] = jnp.dot(h, w2_ref[...], preferred_element_type=jnp.float32) + b2_ref[...]


def _edge2(xsg, xdg, ea_in, w1e, w2, b2, interpret=False):
    e = xsg.shape[0]
    ed = ea_in.shape[1]
    g = e // _BE
    return pl.pallas_call(
        _edge2_body,
        grid=(g,),
        in_specs=[
            pl.BlockSpec((_BE, 64), lambda i: (i, 0)),
            pl.BlockSpec((_BE, 64), lambda i: (i, 0)),
            pl.BlockSpec((_BE, ed), lambda i: (i, 0)),
            pl.BlockSpec((ed, 64), lambda i: (0, 0)),
            pl.BlockSpec((64, 16), lambda i: (0, 0)),
            pl.BlockSpec((1, 16), lambda i: (0, 0)),
        ],
        out_specs=[pl.BlockSpec((_BE, 16), lambda i: (i, 0))],
        out_shape=[jax.ShapeDtypeStruct((e, 16), jnp.float32)],
        interpret=interpret,
    )(xsg, xdg, ea_in, w1e, w2, b2)


def _node1_body(x_ref, p0_ref, p1_ref, degp_ref, batch_ref, w1x_ref,
                w1a_ref, b1_ref, w2_ref, b2_ref, xo_ref, gx_ref):
    deg = jnp.sum(degp_ref[0], axis=0)[:, None]
    agg = (p0_ref[...] + p1_ref[...]) / jnp.maximum(deg, 1.0)
    oh = _onehot_from_ids(batch_ref[0, 0, :])
    h = (jnp.dot(x_ref[...], w1x_ref[...], preferred_element_type=jnp.float32)
         + jnp.dot(agg, w1a_ref[...], preferred_element_type=jnp.float32)
         + b1_ref[...])
    h = jnp.maximum(h, 0.0)
    xo = jnp.dot(h, w2_ref[...], preferred_element_type=jnp.float32) + b2_ref[...]
    xo_ref[...] = xo
    gs = lax.dot_general(oh, xo, (((0,), (0,)), ((), ())),
                         preferred_element_type=jnp.float32, precision=_HI)

    @pl.when(pl.program_id(0) == 0)
    def _():
        gx_ref[...] = jnp.zeros_like(gx_ref)

    gx_ref[...] += gs


def _node1(x, p0, p1, degp, batch_r, w1x, w1a, b1, w2, b2, interpret=False):
    n, d = x.shape
    g = n // _BN
    return pl.pallas_call(
        _node1_body,
        grid=(g,),
        in_specs=[
            pl.BlockSpec((_BN, d), lambda i: (i, 0)),
            pl.BlockSpec((_BN, 16), lambda i: (i, 0)),
            pl.BlockSpec((_BN, 16), lambda i: (i, 0)),
            pl.BlockSpec((1, _NW, _BN), lambda i: (i, 0, 0)),
            pl.BlockSpec((1, 1, _BN), lambda i: (i, 0, 0)),
            pl.BlockSpec((d, 64), lambda i: (0, 0)),
            pl.BlockSpec((16, 64), lambda i: (0, 0)),
            pl.BlockSpec((1, 64), lambda i: (0, 0)),
            pl.BlockSpec((64, d), lambda i: (0, 0)),
            pl.BlockSpec((1, d), lambda i: (0, 0)),
        ],
        out_specs=[
            pl.BlockSpec((_BN, d), lambda i: (i, 0)),
            pl.BlockSpec((_NB, d), lambda i: (0, 0)),
        ],
        out_shape=[
            jax.ShapeDtypeStruct((n, d), jnp.float32),
            jax.ShapeDtypeStruct((_NB, d), jnp.float32),
        ],
        interpret=interpret,
    )(x, p0, p1, degp, batch_r, w1x, w1a, b1, w2, b2)


def _node2_body(x_ref, p0_ref, p1_ref, degp_ref, batch_ref, w1x_ref,
                w1a_ref, uwn_ref, b1_ref, w2_ref, b2_ref, gx_ref):
    deg = jnp.sum(degp_ref[0], axis=0)[:, None]
    agg = (p0_ref[...] + p1_ref[...]) / jnp.maximum(deg, 1.0)
    oh = _onehot_from_ids(batch_ref[0, 0, :])
    h = (jnp.dot(x_ref[...], w1x_ref[...], preferred_element_type=jnp.float32)
         + jnp.dot(agg, w1a_ref[...], preferred_element_type=jnp.float32)
         + jnp.dot(oh, uwn_ref[...], preferred_element_type=jnp.float32,
                   precision=_HI)
         + b1_ref[...])
    h = jnp.maximum(h, 0.0)
    xo = jnp.dot(h, w2_ref[...], preferred_element_type=jnp.float32) + b2_ref[...]
    gs = lax.dot_general(oh, xo, (((0,), (0,)), ((), ())),
                         preferred_element_type=jnp.float32, precision=_HI)

    @pl.when(pl.program_id(0) == 0)
    def _():
        gx_ref[...] = jnp.zeros_like(gx_ref)

    gx_ref[...] += gs


def _node2(x, p0, p1, degp, batch_r, w1x, w1a, uwn, b1, w2, b2, interpret=False):
    n, d = x.shape
    g = n // _BN
    return pl.pallas_call(
        _node2_body,
        grid=(g,),
        in_specs=[
            pl.BlockSpec((_BN, d), lambda i: (i, 0)),
            pl.BlockSpec((_BN, 16), lambda i: (i, 0)),
            pl.BlockSpec((_BN, 16), lambda i: (i, 0)),
            pl.BlockSpec((1, _NW, _BN), lambda i: (i, 0, 0)),
            pl.BlockSpec((1, 1, _BN), lambda i: (i, 0, 0)),
            pl.BlockSpec((d, 64), lambda i: (0, 0)),
            pl.BlockSpec((16, 64), lambda i: (0, 0)),
            pl.BlockSpec((_NB, 64), lambda i: (0, 0)),
            pl.BlockSpec((1, 64), lambda i: (0, 0)),
            pl.BlockSpec((64, d), lambda i: (0, 0)),
            pl.BlockSpec((1, d), lambda i: (0, 0)),
        ],
        out_specs=[pl.BlockSpec((_NB, d), lambda i: (0, 0))],
        out_shape=[jax.ShapeDtypeStruct((_NB, d), jnp.float32)],
        interpret=interpret,
    )(x, p0, p1, degp, batch_r, w1x, w1a, uwn, b1, w2, b2)


def _glob1_body(cnt_ref, ge_ref, gx_ref, wg_gx_ref, wg_ge_ref, b1_ref, w2_ref,
                b2_ref, we_ref, wn_ref, uwe_ref, uwn_ref):
    ncnt = cnt_ref[0:1, :]
    gx = gx_ref[...] / jnp.maximum(ncnt, 1.0).reshape(_NB, 1)
    ecnt = ge_ref[16:17, :]
    ge = ge_ref[0:16, :] / jnp.maximum(ecnt, 1.0).reshape(_NB, 1)
    h = (jnp.dot(gx, wg_gx_ref[...], preferred_element_type=jnp.float32)
         + jnp.dot(ge, wg_ge_ref[...], preferred_element_type=jnp.float32)
         + b1_ref[...])
    h = jnp.maximum(h, 0.0)
    u1 = jnp.dot(h, w2_ref[...], preferred_element_type=jnp.float32) + b2_ref[...]
    uwe_ref[...] = jnp.dot(u1, we_ref[...], preferred_element_type=jnp.float32)
    uwn_ref[...] = jnp.dot(u1, wn_ref[...], preferred_element_type=jnp.float32)


def _glob1(cnt, ge, gx, wg_gx, wg_ge, b1, w2, b2, we, wn, interpret=False):
    return pl.pallas_call(
        _glob1_body,
        out_shape=[
            jax.ShapeDtypeStruct((_NB, 64), jnp.float32),
            jax.ShapeDtypeStruct((_NB, 64), jnp.float32),
        ],
        interpret=interpret,
    )(cnt, ge, gx, wg_gx, wg_ge, b1, w2, b2, we, wn)


def _bn16(h, g, b):
    m = jnp.mean(h, axis=0, keepdims=True)
    v = jnp.mean((h - m) ** 2, axis=0, keepdims=True)
    return g * (h - m) / jnp.sqrt(v + 1e-5) + b


def _head_body(gx_ref, cnt_ref, aw1_ref, ab1_ref, ag1_ref, abe1_ref, aw2_ref,
               ab2_ref, ow1_ref, ob1_ref, og1_ref, obe1_ref, ow2_ref, ob2_ref,
               og2_ref, obe2_ref, ow3_ref, ob3_ref, act_ref, obj_ref):
    maxn = jnp.max(cnt_ref[0:1, :])
    outputs = gx_ref[...] / maxn
    h = _bn16(jnp.dot(outputs, aw1_ref[...], preferred_element_type=jnp.float32)
              + ab1_ref[...], ag1_ref[...], abe1_ref[...])
    act_ref[...] = jnp.dot(jnp.maximum(h, 0.0), aw2_ref[...],
                           preferred_element_type=jnp.float32) + ab2_ref[...]
    h = jnp.maximum(_bn16(
        jnp.dot(outputs, ow1_ref[...], preferred_element_type=jnp.float32)
        + ob1_ref[...], og1_ref[...], obe1_ref[...]), 0.0)
    h = jnp.maximum(_bn16(
        jnp.dot(h, ow2_ref[...], preferred_element_type=jnp.float32)
        + ob2_ref[...], og2_ref[...], obe2_ref[...]), 0.0)
    obj_ref[...] = jnp.dot(h, ow3_ref[...],
                           preferred_element_type=jnp.float32) + ob3_ref[...]


def _head(gx2, cnt, pa, po, interpret=False):
    args = (gx2, cnt,
            pa["W1"], pa["b1"].reshape(1, -1), pa["g1"].reshape(1, -1),
            pa["be1"].reshape(1, -1), pa["W2"], pa["b2"].reshape(1, -1),
            po["W1"], po["b1"].reshape(1, -1), po["g1"].reshape(1, -1),
            po["be1"].reshape(1, -1), po["W2"], po["b2"].reshape(1, -1),
            po["g2"].reshape(1, -1), po["be2"].reshape(1, -1), po["W3"],
            po["b3"].reshape(1, -1))
    return pl.pallas_call(
        _head_body,
        out_shape=[
            jax.ShapeDtypeStruct((_NB, 32), jnp.float32),
            jax.ShapeDtypeStruct((_NB, 64), jnp.float32),
        ],
        interpret=interpret,
    )(*args)


@functools.lru_cache(maxsize=None)
def _build_sc_gather(n, e):
    mesh = plsc.VectorSubcoreMesh(core_axis_name="c", subcore_axis_name="s")
    epw = e // _NW
    nrow = epw // _W
    ng = nrow // _GB
    grp = _GB * _W

    @functools.partial(
        pl.kernel, mesh=mesh,
        compiler_params=pltpu.CompilerParams(use_tc_tiling_on_sc=False),
        out_type=(jax.ShapeDtypeStruct((e, 64), jnp.float32),
                  jax.ShapeDtypeStruct((e, 64), jnp.float32)),
        scratch_types=[
            pltpu.VMEM((nrow, _W), jnp.int32),
            pltpu.VMEM((nrow, _W), jnp.int32),
            pltpu.VMEM((grp, 64), jnp.float32),
            pltpu.SemaphoreType.DMA,
        ],
    )
    def gk(xs_hbm, xd_hbm, src_hbm, dst_hbm, xsg_hbm, xdg_hbm,
           idxs, idxd, rows, sem):
        wid = lax.axis_index("s") * 2 + lax.axis_index("c")
        tb = wid * nrow
        base = wid * epw
        pltpu.sync_copy(src_hbm.at[pl.ds(tb, nrow)], idxs)
        pltpu.sync_copy(dst_hbm.at[pl.ds(tb, nrow)], idxd)

        def group(g, carry):
            off = base + g * grp
            cps = [pltpu.async_copy(xs_hbm.at[idxs.at[g * _GB + b]],
                                    rows.at[pl.ds(b * _W, _W)], sem)
                   for b in range(_GB)]
            for cp in cps:
                cp.wait()
            pltpu.sync_copy(rows, xsg_hbm.at[pl.ds(off, grp)])
            cps = [pltpu.async_copy(xd_hbm.at[idxd.at[g * _GB + b]],
                                    rows.at[pl.ds(b * _W, _W)], sem)
                   for b in range(_GB)]
            for cp in cps:
                cp.wait()
            pltpu.sync_copy(rows, xdg_hbm.at[pl.ds(off, grp)])
            return carry

        lax.fori_loop(0, ng, group, 0)

    return gk


@functools.lru_cache(maxsize=None)
def _build_sc_scatter(n, e, with_deg):
    mesh = plsc.VectorSubcoreMesh(core_axis_name="c", subcore_axis_name="s")
    epw = e // _NW
    nrow = epw // _W
    ng = nrow // _GB
    grp = _GB * _W
    nzw = 10
    rps = n // nzw
    nv = n // 16
    ev = epw // 16

    outs = [jax.ShapeDtypeStruct((2, n, 16), jnp.float32)]
    scratch = [
        pltpu.VMEM((nrow, _W), jnp.int32),
        pltpu.VMEM((grp, 16), jnp.float32),
        pltpu.VMEM_SHARED((n, 16), jnp.float32),
    ]
    if with_deg:
        outs.append(jax.ShapeDtypeStruct((_NW * n,), jnp.float32))
        scratch += [
            pltpu.VMEM((epw,), jnp.int32),
            pltpu.VMEM((n,), jnp.float32),
        ]

    def body(ea_hbm, dst_hbm, dstf_hbm, zeros_hbm, agg_hbm, deg_hbm,
             idxd, rows, accum, dflat, dacc):
        cid = lax.axis_index("c")
        sid = lax.axis_index("s")
        wid = sid * 2 + cid

        @pl.when(sid < nzw)
        def _():
            pltpu.sync_copy(zeros_hbm, accum.at[pl.ds(sid * rps, rps)])

        plsc.subcore_barrier()
        tb = wid * nrow
        base = wid * epw
        pltpu.sync_copy(dst_hbm.at[pl.ds(tb, nrow)], idxd)

        if with_deg:
            pltpu.sync_copy(dstf_hbm.at[pl.ds(base, epw)], dflat)
            zv = jnp.zeros((16,), jnp.float32)

            def zloop(j, c):
                dacc[pl.ds(j * 16, 16)] = zv
                return c

            lax.fori_loop(0, nv, zloop, 0)
            ones = jnp.full((16,), 1.0, jnp.float32)

            def dloop(j, c):
                idx = dflat[pl.ds(j * 16, 16)]
                plsc.addupdate_scatter(dacc, [idx], ones)
                return c

            lax.fori_loop(0, ev, dloop, 0)
            for blk in range(n // _BN):
                pltpu.sync_copy(
                    dacc.at[pl.ds(blk * _BN, _BN)],
                    deg_hbm.at[pl.ds(blk * _NW * _BN + wid * _BN, _BN)])

        def group(g, carry):
            pltpu.sync_copy(ea_hbm.at[pl.ds(base + g * grp, grp)], rows)
            for b in range(_GB):
                pltpu.sync_copy(rows.at[pl.ds(b * _W, _W)],
                                accum.at[idxd.at[g * _GB + b]], add=True)
            return carry

        lax.fori_loop(0, ng, group, 0)
        plsc.subcore_barrier()

        @pl.when(sid < nzw)
        def _():
            pltpu.sync_copy(accum.at[pl.ds(sid * rps, rps)],
                            agg_hbm.at[cid, pl.ds(sid * rps, rps)])

    if with_deg:
        def sk(ea_hbm, dst_hbm, dstf_hbm, zeros_hbm, agg_hbm, deg_hbm,
               idxd, rows, accum, dflat, dacc):
            body(ea_hbm, dst_hbm, dstf_hbm, zeros_hbm, agg_hbm, deg_hbm,
                 idxd, rows, accum, dflat, dacc)
    else:
        def sk(ea_hbm, dst_hbm, zeros_hbm, agg_hbm, idxd, rows, accum):
            body(ea_hbm, dst_hbm, None, zeros_hbm, agg_hbm, None,
                 idxd, rows, accum, None, None)

    return functools.partial(
        pl.kernel, mesh=mesh, out_type=tuple(outs),
        compiler_params=pltpu.CompilerParams(use_tc_tiling_on_sc=False,
                                             needs_layout_passes=False),
        scratch_types=scratch)(sk)


def kernel(x, edge_index, edge_attr, batch, params):
    n, d = x.shape
    e = edge_index.shape[1]
    src = edge_index[0].astype(jnp.int32)
    dst = edge_index[1].astype(jnp.int32)
    src2d = src.reshape(e // _W, _W)
    dst2d = dst.reshape(e // _W, _W)
    src_r = src.reshape(e // _BE, 1, _BE)
    batch_r = batch.astype(jnp.int32).reshape(n // _BN, 1, _BN)
    zeros_np = jnp.zeros((n // 10, 16), jnp.float32)

    p1, p2 = params["gnn1"], params["gnn2"]
    pe1, pn1, pg1 = p1["edge"], p1["node"], p1["glob"]
    pe2, pn2 = p2["edge"], p2["node"]

    gather = _build_sc_gather(n, e)
    scatter1 = _build_sc_scatter(n, e, True)
    scatter2 = _build_sc_scatter(n, e, False)

    xs1, xd1, cnt = _prep1(x, pe1["W1"][:d], pe1["W1"][d:2 * d],
                           pe1["b1"].reshape(1, -1), batch_r)
    xsg1, xdg1 = gather(xs1, xd1, src2d, dst2d)
    ea1, ge = _edge1(xsg1, xdg1, edge_attr, src_r, cnt,
                     pe1["W1"][2 * d:2 * d + 16], pe1["W2"],
                     pe1["b2"].reshape(1, -1))
    aggp, degf = scatter1(ea1, dst2d, dst, zeros_np)
    degp = degf.reshape(n // _BN, _NW, _BN)
    x1, gx1 = _node1(x, aggp[0], aggp[1], degp, batch_r,
                     pn1["W1"][:d], pn1["W1"][d:d + 16],
                     pn1["b1"].reshape(1, -1), pn1["W2"],
                     pn1["b2"].reshape(1, -1))
    uwe2, uwn2 = _glob1(cnt, ge, gx1, pg1["W1"][16:16 + d],
                        pg1["W1"][16 + d:], pg1["b1"].reshape(1, -1),
                        pg1["W2"], pg1["b2"].reshape(1, -1),
                        pe2["W1"][2 * d + 16:], pn2["W1"][d + 16:])

    xs2, xd2 = _prep2(x1, pe2["W1"][:d], pe2["W1"][d:2 * d],
                      pe2["b1"].reshape(1, -1), batch_r, uwe2)
    xsg2, xdg2 = gather(xs2, xd2, src2d, dst2d)
    (ea2,) = _edge2(xsg2, xdg2, ea1, pe2["W1"][2 * d:2 * d + 16], pe2["W2"],
                    pe2["b2"].reshape(1, -1))
    res2 = scatter2(ea2, dst2d, zeros_np)
    aggp2 = res2[0] if isinstance(res2, (tuple, list)) else res2
    (gx2,) = _node2(x1, aggp2[0], aggp2[1], degp, batch_r,
                    pn2["W1"][:d], pn2["W1"][d:d + 16], uwn2,
                    pn2["b1"].reshape(1, -1), pn2["W2"],
                    pn2["b2"].reshape(1, -1))
    act, obj = _head(gx2, cnt, params["action"], params["object"])
    return act, obj


# speedup vs baseline: 6.7239x; 1.1185x over previous
import functools

import jax
import jax.numpy as jnp
from jax import lax
from jax.experimental import pallas as pl
from jax.experimental.pallas import tpu as pltpu
from jax.experimental.pallas import tpu_sc as plsc

_BN = 1000
_BE = 6400
_W = 125
_GB = 8
_NB = 16
_NW = 32

_HI = lax.Precision.HIGHEST


def _iota16():
    return lax.broadcasted_iota(jnp.int32, (1, _NB), 1)


def _onehot_from_ids(ids):
    return (ids[:, None] == _iota16()).astype(jnp.float32)


def _onehot_t_from_src(src_f, counts_row):
    row = lax.broadcasted_iota(jnp.int32, (_NB, _NB), 0)
    col = lax.broadcasted_iota(jnp.int32, (_NB, _NB), 1)
    lt = (row < col).astype(jnp.float32)
    cum_excl = jnp.dot(counts_row, lt, preferred_element_type=jnp.float32,
                       precision=_HI)
    upper = cum_excl + counts_row
    s = src_f[None, :]
    return ((s >= cum_excl.reshape(_NB, 1)) & (s < upper.reshape(_NB, 1))
            ).astype(jnp.float32)


def _prep1_body(x_ref, ws_ref, wd_ref, b1_ref, batch_ref, xs_ref, xd_ref, cnt_ref):
    x = x_ref[...]
    xs_ref[...] = jnp.dot(x, ws_ref[...], preferred_element_type=jnp.float32) + b1_ref[...]
    xd_ref[...] = jnp.dot(x, wd_ref[...], preferred_element_type=jnp.float32)
    oh = _onehot_from_ids(batch_ref[0, 0, :])
    cnt = jnp.sum(oh, axis=0)

    @pl.when(pl.program_id(0) == 0)
    def _():
        cnt_ref[...] = jnp.zeros_like(cnt_ref)

    cnt_ref[...] += jnp.concatenate(
        [cnt[None, :], jnp.zeros((7, _NB), jnp.float32)], axis=0)


def _prep1(x, ws, wd, b1, batch_r, interpret=False):
    n, d = x.shape
    g = n // _BN
    return pl.pallas_call(
        _prep1_body,
        grid=(g,),
        in_specs=[
            pl.BlockSpec((_BN, d), lambda i: (i, 0)),
            pl.BlockSpec((d, 64), lambda i: (0, 0)),
            pl.BlockSpec((d, 64), lambda i: (0, 0)),
            pl.BlockSpec((1, 64), lambda i: (0, 0)),
            pl.BlockSpec((1, 1, _BN), lambda i: (i, 0, 0)),
        ],
        out_specs=[
            pl.BlockSpec((_BN, 64), lambda i: (i, 0)),
            pl.BlockSpec((_BN, 64), lambda i: (i, 0)),
            pl.BlockSpec((8, _NB), lambda i: (0, 0)),
        ],
        out_shape=[
            jax.ShapeDtypeStruct((n, 64), jnp.float32),
            jax.ShapeDtypeStruct((n, 64), jnp.float32),
            jax.ShapeDtypeStruct((8, _NB), jnp.float32),
        ],
        interpret=interpret,
    )(x, ws, wd, b1, batch_r)


def _prep2_body(x_ref, ws_ref, wd_ref, b1_ref, batch_ref, uwe_ref, xs_ref, xd_ref):
    x = x_ref[...]
    oh = _onehot_from_ids(batch_ref[0, 0, :])
    xs_ref[...] = (jnp.dot(x, ws_ref[...], preferred_element_type=jnp.float32)
                   + jnp.dot(oh, uwe_ref[...], preferred_element_type=jnp.float32,
                             precision=_HI)
                   + b1_ref[...])
    xd_ref[...] = jnp.dot(x, wd_ref[...], preferred_element_type=jnp.float32)


def _prep2(x, ws, wd, b1, batch_r, uwe, interpret=False):
    n, d = x.shape
    g = n // _BN
    return pl.pallas_call(
        _prep2_body,
        grid=(g,),
        in_specs=[
            pl.BlockSpec((_BN, d), lambda i: (i, 0)),
            pl.BlockSpec((d, 64), lambda i: (0, 0)),
            pl.BlockSpec((d, 64), lambda i: (0, 0)),
            pl.BlockSpec((1, 64), lambda i: (0, 0)),
            pl.BlockSpec((1, 1, _BN), lambda i: (i, 0, 0)),
            pl.BlockSpec((_NB, 64), lambda i: (0, 0)),
        ],
        out_specs=[
            pl.BlockSpec((_BN, 64), lambda i: (i, 0)),
            pl.BlockSpec((_BN, 64), lambda i: (i, 0)),
        ],
        out_shape=[
            jax.ShapeDtypeStruct((n, 64), jnp.float32),
            jax.ShapeDtypeStruct((n, 64), jnp.float32),
        ],
        interpret=interpret,
    )(x, ws, wd, b1, batch_r, uwe)


def _edge1_body(xsg_ref, xdg_ref, ea_ref, src_ref, cnt_ref, w1e_ref, w2_ref,
                b2_ref, eo_ref, ge_ref):
    oht = _onehot_t_from_src(src_ref[0, 0, :].astype(jnp.float32), cnt_ref[0:1, :])
    h = xsg_ref[...] + xdg_ref[...] + lax.dot_general(
        ea_ref[...], w1e_ref[...], (((0,), (0,)), ((), ())),
        preferred_element_type=jnp.float32)
    h = jnp.maximum(h, 0.0)
    ea = jnp.dot(h, w2_ref[...], preferred_element_type=jnp.float32) + b2_ref[...]
    eo_ref[...] = ea
    gs = jnp.dot(oht, ea, preferred_element_type=jnp.float32, precision=_HI)
    ecnt = jnp.sum(oht, axis=1)
    upd = jnp.concatenate(
        [gs, ecnt[None, :], jnp.zeros((7, _NB), jnp.float32)], axis=0)

    @pl.when(pl.program_id(0) == 0)
    def _():
        ge_ref[...] = jnp.zeros_like(ge_ref)

    ge_ref[...] += upd


def _edge1(xsg, xdg, ea_in_t, src_r, cnt, w1e, w2, b2, interpret=False):
    e = xsg.shape[0]
    ed = ea_in_t.shape[0]
    g = e // _BE
    return pl.pallas_call(
        _edge1_body,
        grid=(g,),
        in_specs=[
            pl.BlockSpec((_BE, 64), lambda i: (i, 0)),
            pl.BlockSpec((_BE, 64), lambda i: (i, 0)),
            pl.BlockSpec((ed, _BE), lambda i: (0, i)),
            pl.BlockSpec((1, 1, _BE), lambda i: (i, 0, 0)),
            pl.BlockSpec((8, _NB), lambda i: (0, 0)),
            pl.BlockSpec((ed, 64), lambda i: (0, 0)),
            pl.BlockSpec((64, 16), lambda i: (0, 0)),
            pl.BlockSpec((1, 16), lambda i: (0, 0)),
        ],
        out_specs=[
            pl.BlockSpec((_BE, 16), lambda i: (i, 0)),
            pl.BlockSpec((24, _NB), lambda i: (0, 0)),
        ],
        out_shape=[
            jax.ShapeDtypeStruct((e, 16), jnp.float32),
            jax.ShapeDtypeStruct((24, _NB), jnp.float32),
        ],
        interpret=interpret,
    )(xsg, xdg, ea_in_t, src_r, cnt, w1e, w2, b2)


def _edge2_body(xsg_ref, xdg_ref, ea_ref, w1e_ref, w2_ref, b2_ref, eo_ref):
    h = xsg_ref[...] + xdg_ref[...] + jnp.dot(
        ea_ref[...], w1e_ref[...], preferred_element_type=jnp.float32)
    h = jnp.maximum(h, 0.0)
    eo_ref[...] = jnp.dot(h, w2_ref[...], preferred_element_type=jnp.float32) + b2_ref[...]


def _edge2(xsg, xdg, ea_in, w1e, w2, b2, interpret=False):
    e = xsg.shape[0]
    ed = ea_in.shape[1]
    g = e // _BE
    return pl.pallas_call(
        _edge2_body,
        grid=(g,),
        in_specs=[
            pl.BlockSpec((_BE, 64), lambda i: (i, 0)),
            pl.BlockSpec((_BE, 64), lambda i: (i, 0)),
            pl.BlockSpec((_BE, ed), lambda i: (i, 0)),
            pl.BlockSpec((ed, 64), lambda i: (0, 0)),
            pl.BlockSpec((64, 16), lambda i: (0, 0)),
            pl.BlockSpec((1, 16), lambda i: (0, 0)),
        ],
        out_specs=[pl.BlockSpec((_BE, 16), lambda i: (i, 0))],
        out_shape=[jax.ShapeDtypeStruct((e, 16), jnp.float32)],
        interpret=interpret,
    )(xsg, xdg, ea_in, w1e, w2, b2)


def _node1_body(x_ref, p0_ref, p1_ref, degp_ref, batch_ref, w1x_ref,
                w1a_ref, b1_ref, w2_ref, b2_ref, xo_ref, gx_ref):
    deg = jnp.sum(degp_ref[0], axis=0)[:, None]
    agg = (p0_ref[...] + p1_ref[...]) / jnp.maximum(deg, 1.0)
    oh = _onehot_from_ids(batch_ref[0, 0, :])
    h = (jnp.dot(x_ref[...], w1x_ref[...], preferred_element_type=jnp.float32)
         + jnp.dot(agg, w1a_ref[...], preferred_element_type=jnp.float32)
         + b1_ref[...])
    h = jnp.maximum(h, 0.0)
    xo = jnp.dot(h, w2_ref[...], preferred_element_type=jnp.float32) + b2_ref[...]
    xo_ref[...] = xo
    gs = lax.dot_general(oh, xo, (((0,), (0,)), ((), ())),
                         preferred_element_type=jnp.float32, precision=_HI)

    @pl.when(pl.program_id(0) == 0)
    def _():
        gx_ref[...] = jnp.zeros_like(gx_ref)

    gx_ref[...] += gs


def _node1(x, p0, p1, degp, batch_r, w1x, w1a, b1, w2, b2, interpret=False):
    n, d = x.shape
    g = n // _BN
    return pl.pallas_call(
        _node1_body,
        grid=(g,),
        in_specs=[
            pl.BlockSpec((_BN, d), lambda i: (i, 0)),
            pl.BlockSpec((_BN, 16), lambda i: (i, 0)),
            pl.BlockSpec((_BN, 16), lambda i: (i, 0)),
            pl.BlockSpec((1, _NW, _BN), lambda i: (i, 0, 0)),
            pl.BlockSpec((1, 1, _BN), lambda i: (i, 0, 0)),
            pl.BlockSpec((d, 64), lambda i: (0, 0)),
            pl.BlockSpec((16, 64), lambda i: (0, 0)),
            pl.BlockSpec((1, 64), lambda i: (0, 0)),
            pl.BlockSpec((64, d), lambda i: (0, 0)),
            pl.BlockSpec((1, d), lambda i: (0, 0)),
        ],
        out_specs=[
            pl.BlockSpec((_BN, d), lambda i: (i, 0)),
            pl.BlockSpec((_NB, d), lambda i: (0, 0)),
        ],
        out_shape=[
            jax.ShapeDtypeStruct((n, d), jnp.float32),
            jax.ShapeDtypeStruct((_NB, d), jnp.float32),
        ],
        interpret=interpret,
    )(x, p0, p1, degp, batch_r, w1x, w1a, b1, w2, b2)


def _node2_body(x_ref, p0_ref, p1_ref, degp_ref, batch_ref, w1x_ref,
                w1a_ref, uwn_ref, b1_ref, w2_ref, b2_ref, gx_ref):
    deg = jnp.sum(degp_ref[0], axis=0)[:, None]
    agg = (p0_ref[...] + p1_ref[...]) / jnp.maximum(deg, 1.0)
    oh = _onehot_from_ids(batch_ref[0, 0, :])
    h = (jnp.dot(x_ref[...], w1x_ref[...], preferred_element_type=jnp.float32)
         + jnp.dot(agg, w1a_ref[...], preferred_element_type=jnp.float32)
         + jnp.dot(oh, uwn_ref[...], preferred_element_type=jnp.float32,
                   precision=_HI)
         + b1_ref[...])
    h = jnp.maximum(h, 0.0)
    xo = jnp.dot(h, w2_ref[...], preferred_element_type=jnp.float32) + b2_ref[...]
    gs = lax.dot_general(oh, xo, (((0,), (0,)), ((), ())),
                         preferred_element_type=jnp.float32, precision=_HI)

    @pl.when(pl.program_id(0) == 0)
    def _():
        gx_ref[...] = jnp.zeros_like(gx_ref)

    gx_ref[...] += gs


def _node2(x, p0, p1, degp, batch_r, w1x, w1a, uwn, b1, w2, b2, interpret=False):
    n, d = x.shape
    g = n // _BN
    return pl.pallas_call(
        _node2_body,
        grid=(g,),
        in_specs=[
            pl.BlockSpec((_BN, d), lambda i: (i, 0)),
            pl.BlockSpec((_BN, 16), lambda i: (i, 0)),
            pl.BlockSpec((_BN, 16), lambda i: (i, 0)),
            pl.BlockSpec((1, _NW, _BN), lambda i: (i, 0, 0)),
            pl.BlockSpec((1, 1, _BN), lambda i: (i, 0, 0)),
            pl.BlockSpec((d, 64), lambda i: (0, 0)),
            pl.BlockSpec((16, 64), lambda i: (0, 0)),
            pl.BlockSpec((_NB, 64), lambda i: (0, 0)),
            pl.BlockSpec((1, 64), lambda i: (0, 0)),
            pl.BlockSpec((64, d), lambda i: (0, 0)),
            pl.BlockSpec((1, d), lambda i: (0, 0)),
        ],
        out_specs=[pl.BlockSpec((_NB, d), lambda i: (0, 0))],
        out_shape=[jax.ShapeDtypeStruct((_NB, d), jnp.float32)],
        interpret=interpret,
    )(x, p0, p1, degp, batch_r, w1x, w1a, uwn, b1, w2, b2)


def _glob1_body(cnt_ref, ge_ref, gx_ref, wg_gx_ref, wg_ge_ref, b1_ref, w2_ref,
                b2_ref, we_ref, wn_ref, uwe_ref, uwn_ref):
    ncnt = cnt_ref[0:1, :]
    gx = gx_ref[...] / jnp.maximum(ncnt, 1.0).reshape(_NB, 1)
    ecnt = ge_ref[16:17, :]
    ge = ge_ref[0:16, :] / jnp.maximum(ecnt, 1.0).reshape(_NB, 1)
    h = (jnp.dot(gx, wg_gx_ref[...], preferred_element_type=jnp.float32)
         + jnp.dot(ge, wg_ge_ref[...], preferred_element_type=jnp.float32)
         + b1_ref[...])
    h = jnp.maximum(h, 0.0)
    u1 = jnp.dot(h, w2_ref[...], preferred_element_type=jnp.float32) + b2_ref[...]
    uwe_ref[...] = jnp.dot(u1, we_ref[...], preferred_element_type=jnp.float32)
    uwn_ref[...] = jnp.dot(u1, wn_ref[...], preferred_element_type=jnp.float32)


def _glob1(cnt, ge, gx, wg_gx, wg_ge, b1, w2, b2, we, wn, interpret=False):
    return pl.pallas_call(
        _glob1_body,
        out_shape=[
            jax.ShapeDtypeStruct((_NB, 64), jnp.float32),
            jax.ShapeDtypeStruct((_NB, 64), jnp.float32),
        ],
        interpret=interpret,
    )(cnt, ge, gx, wg_gx, wg_ge, b1, w2, b2, we, wn)


def _bn16(h, g, b):
    m = jnp.mean(h, axis=0, keepdims=True)
    v = jnp.mean((h - m) ** 2, axis=0, keepdims=True)
    return g * (h - m) / jnp.sqrt(v + 1e-5) + b


def _head_body(gx_ref, cnt_ref, aw1_ref, ab1_ref, ag1_ref, abe1_ref, aw2_ref,
               ab2_ref, ow1_ref, ob1_ref, og1_ref, obe1_ref, ow2_ref, ob2_ref,
               og2_ref, obe2_ref, ow3_ref, ob3_ref, act_ref, obj_ref):
    maxn = jnp.max(cnt_ref[0:1, :])
    outputs = gx_ref[...] / maxn
    h = _bn16(jnp.dot(outputs, aw1_ref[...], preferred_element_type=jnp.float32)
              + ab1_ref[...], ag1_ref[...], abe1_ref[...])
    act_ref[...] = jnp.dot(jnp.maximum(h, 0.0), aw2_ref[...],
                           preferred_element_type=jnp.float32) + ab2_ref[...]
    h = jnp.maximum(_bn16(
        jnp.dot(outputs, ow1_ref[...], preferred_element_type=jnp.float32)
        + ob1_ref[...], og1_ref[...], obe1_ref[...]), 0.0)
    h = jnp.maximum(_bn16(
        jnp.dot(h, ow2_ref[...], preferred_element_type=jnp.float32)
        + ob2_ref[...], og2_ref[...], obe2_ref[...]), 0.0)
    obj_ref[...] = jnp.dot(h, ow3_ref[...],
                           preferred_element_type=jnp.float32) + ob3_ref[...]


def _head(gx2, cnt, pa, po, interpret=False):
    args = (gx2, cnt,
            pa["W1"], pa["b1"].reshape(1, -1), pa["g1"].reshape(1, -1),
            pa["be1"].reshape(1, -1), pa["W2"], pa["b2"].reshape(1, -1),
            po["W1"], po["b1"].reshape(1, -1), po["g1"].reshape(1, -1),
            po["be1"].reshape(1, -1), po["W2"], po["b2"].reshape(1, -1),
            po["g2"].reshape(1, -1), po["be2"].reshape(1, -1), po["W3"],
            po["b3"].reshape(1, -1))
    return pl.pallas_call(
        _head_body,
        out_shape=[
            jax.ShapeDtypeStruct((_NB, 32), jnp.float32),
            jax.ShapeDtypeStruct((_NB, 64), jnp.float32),
        ],
        interpret=interpret,
    )(*args)


@functools.lru_cache(maxsize=None)
def _build_sc_gather(n, e):
    mesh = plsc.VectorSubcoreMesh(core_axis_name="c", subcore_axis_name="s")
    epw = e // _NW
    nrow = epw // _W
    ng = nrow // _GB
    grp = _GB * _W

    @functools.partial(
        pl.kernel, mesh=mesh,
        compiler_params=pltpu.CompilerParams(use_tc_tiling_on_sc=False),
        out_type=(jax.ShapeDtypeStruct((e, 64), jnp.float32),
                  jax.ShapeDtypeStruct((e, 64), jnp.float32)),
        scratch_types=[
            pltpu.VMEM((nrow, _W), jnp.int32),
            pltpu.VMEM((nrow, _W), jnp.int32),
            pltpu.VMEM((grp, 64), jnp.float32),
            pltpu.SemaphoreType.DMA,
        ],
    )
    def gk(xs_hbm, xd_hbm, src_hbm, dst_hbm, xsg_hbm, xdg_hbm,
           idxs, idxd, rows, sem):
        wid = lax.axis_index("s") * 2 + lax.axis_index("c")
        tb = wid * nrow
        base = wid * epw
        pltpu.sync_copy(src_hbm.at[pl.ds(tb, nrow)], idxs)
        pltpu.sync_copy(dst_hbm.at[pl.ds(tb, nrow)], idxd)

        def group(g, carry):
            off = base + g * grp
            cps = [pltpu.async_copy(xs_hbm.at[idxs.at[g * _GB + b]],
                                    rows.at[pl.ds(b * _W, _W)], sem)
                   for b in range(_GB)]
            for cp in cps:
                cp.wait()
            pltpu.sync_copy(rows, xsg_hbm.at[pl.ds(off, grp)])
            cps = [pltpu.async_copy(xd_hbm.at[idxd.at[g * _GB + b]],
                                    rows.at[pl.ds(b * _W, _W)], sem)
                   for b in range(_GB)]
            for cp in cps:
                cp.wait()
            pltpu.sync_copy(rows, xdg_hbm.at[pl.ds(off, grp)])
            return carry

        lax.fori_loop(0, ng, group, 0)

    return gk


@functools.lru_cache(maxsize=None)
def _build_sc_scatter(n, e, with_deg):
    mesh = plsc.VectorSubcoreMesh(core_axis_name="c", subcore_axis_name="s")
    epw = e // _NW
    nrow = epw // _W
    ng = nrow // _GB
    grp = _GB * _W
    nzw = 10
    rps = n // nzw
    nv = n // 16
    ev = epw // 16

    outs = [jax.ShapeDtypeStruct((2, n, 16), jnp.float32)]
    scratch = [
        pltpu.VMEM((nrow, _W), jnp.int32),
        pltpu.VMEM((grp, 16), jnp.float32),
        pltpu.VMEM_SHARED((n, 16), jnp.float32),
    ]
    if with_deg:
        outs.append(jax.ShapeDtypeStruct((_NW * n,), jnp.float32))
        scratch += [
            pltpu.VMEM((epw,), jnp.int32),
            pltpu.VMEM((n,), jnp.float32),
        ]

    def body(ea_hbm, dst_hbm, dstf_hbm, zeros_hbm, agg_hbm, deg_hbm,
             idxd, rows, accum, dflat, dacc):
        cid = lax.axis_index("c")
        sid = lax.axis_index("s")
        wid = sid * 2 + cid

        @pl.when(sid < nzw)
        def _():
            pltpu.sync_copy(zeros_hbm, accum.at[pl.ds(sid * rps, rps)])

        plsc.subcore_barrier()
        tb = wid * nrow
        base = wid * epw
        pltpu.sync_copy(dst_hbm.at[pl.ds(tb, nrow)], idxd)

        if with_deg:
            pltpu.sync_copy(dstf_hbm.at[pl.ds(base, epw)], dflat)
            zv = jnp.zeros((16,), jnp.float32)

            def zloop(j, c):
                dacc[pl.ds(j * 16, 16)] = zv
                return c

            lax.fori_loop(0, nv, zloop, 0)
            ones = jnp.full((16,), 1.0, jnp.float32)

            def dloop(j, c):
                idx = dflat[pl.ds(j * 16, 16)]
                plsc.addupdate_scatter(dacc, [idx], ones)
                return c

            lax.fori_loop(0, ev, dloop, 0)
            for blk in range(n // _BN):
                pltpu.sync_copy(
                    dacc.at[pl.ds(blk * _BN, _BN)],
                    deg_hbm.at[pl.ds(blk * _NW * _BN + wid * _BN, _BN)])

        def group(g, carry):
            pltpu.sync_copy(ea_hbm.at[pl.ds(base + g * grp, grp)], rows)
            for b in range(_GB):
                pltpu.sync_copy(rows.at[pl.ds(b * _W, _W)],
                                accum.at[idxd.at[g * _GB + b]], add=True)
            return carry

        lax.fori_loop(0, ng, group, 0)
        plsc.subcore_barrier()

        @pl.when(sid < nzw)
        def _():
            pltpu.sync_copy(accum.at[pl.ds(sid * rps, rps)],
                            agg_hbm.at[cid, pl.ds(sid * rps, rps)])

    if with_deg:
        def sk(ea_hbm, dst_hbm, dstf_hbm, zeros_hbm, agg_hbm, deg_hbm,
               idxd, rows, accum, dflat, dacc):
            body(ea_hbm, dst_hbm, dstf_hbm, zeros_hbm, agg_hbm, deg_hbm,
                 idxd, rows, accum, dflat, dacc)
    else:
        def sk(ea_hbm, dst_hbm, zeros_hbm, agg_hbm, idxd, rows, accum):
            body(ea_hbm, dst_hbm, None, zeros_hbm, agg_hbm, None,
                 idxd, rows, accum, None, None)

    return functools.partial(
        pl.kernel, mesh=mesh, out_type=tuple(outs),
        compiler_params=pltpu.CompilerParams(use_tc_tiling_on_sc=False,
                                             needs_layout_passes=False),
        scratch_types=scratch)(sk)


def kernel(x, edge_index, edge_attr, batch, params):
    n, d = x.shape
    e = edge_index.shape[1]
    src = edge_index[0].astype(jnp.int32)
    dst = edge_index[1].astype(jnp.int32)
    src2d = src.reshape(e // _W, _W)
    dst2d = dst.reshape(e // _W, _W)
    src_r = src.reshape(e // _BE, 1, _BE)
    batch_r = batch.astype(jnp.int32).reshape(n // _BN, 1, _BN)
    zeros_np = jnp.zeros((n // 10, 16), jnp.float32)

    p1, p2 = params["gnn1"], params["gnn2"]
    pe1, pn1, pg1 = p1["edge"], p1["node"], p1["glob"]
    pe2, pn2 = p2["edge"], p2["node"]

    gather = _build_sc_gather(n, e)
    scatter1 = _build_sc_scatter(n, e, True)
    scatter2 = _build_sc_scatter(n, e, False)

    xs1, xd1, cnt = _prep1(x, pe1["W1"][:d], pe1["W1"][d:2 * d],
                           pe1["b1"].reshape(1, -1), batch_r)
    xsg1, xdg1 = gather(xs1, xd1, src2d, dst2d)
    ea1, ge = _edge1(xsg1, xdg1, edge_attr.T, src_r, cnt,
                     pe1["W1"][2 * d:2 * d + 16], pe1["W2"],
                     pe1["b2"].reshape(1, -1))
    aggp, degf = scatter1(ea1, dst2d, dst, zeros_np)
    degp = degf.reshape(n // _BN, _NW, _BN)
    x1, gx1 = _node1(x, aggp[0], aggp[1], degp, batch_r,
                     pn1["W1"][:d], pn1["W1"][d:d + 16],
                     pn1["b1"].reshape(1, -1), pn1["W2"],
                     pn1["b2"].reshape(1, -1))
    uwe2, uwn2 = _glob1(cnt, ge, gx1, pg1["W1"][16:16 + d],
                        pg1["W1"][16 + d:], pg1["b1"].reshape(1, -1),
                        pg1["W2"], pg1["b2"].reshape(1, -1),
                        pe2["W1"][2 * d + 16:], pn2["W1"][d + 16:])

    xs2, xd2 = _prep2(x1, pe2["W1"][:d], pe2["W1"][d:2 * d],
                      pe2["b1"].reshape(1, -1), batch_r, uwe2)
    xsg2, xdg2 = gather(xs2, xd2, src2d, dst2d)
    (ea2,) = _edge2(xsg2, xdg2, ea1, pe2["W1"][2 * d:2 * d + 16], pe2["W2"],
                    pe2["b2"].reshape(1, -1))
    res2 = scatter2(ea2, dst2d, zeros_np)
    aggp2 = res2[0] if isinstance(res2, (tuple, list)) else res2
    (gx2,) = _node2(x1, aggp2[0], aggp2[1], degp, batch_r,
                    pn2["W1"][:d], pn2["W1"][d:d + 16], uwn2,
                    pn2["b1"].reshape(1, -1), pn2["W2"],
                    pn2["b2"].reshape(1, -1))
    act, obj = _head(gx2, cnt, params["action"], params["object"])
    return act, obj


# speedup vs baseline: 7.1842x; 1.0685x over previous
import functools

import jax
import jax.numpy as jnp
from jax import lax
from jax.experimental import pallas as pl
from jax.experimental.pallas import tpu as pltpu
from jax.experimental.pallas import tpu_sc as plsc

_BN = 1000
_BE = 6400
_W = 125
_GB = 8
_NB = 16
_NW = 32

_HI = lax.Precision.HIGHEST


def _iota16():
    return lax.broadcasted_iota(jnp.int32, (1, _NB), 1)


def _onehot_from_ids(ids):
    return (ids[:, None] == _iota16()).astype(jnp.float32)


def _onehot_t_from_src(src_f, counts_row):
    row = lax.broadcasted_iota(jnp.int32, (_NB, _NB), 0)
    col = lax.broadcasted_iota(jnp.int32, (_NB, _NB), 1)
    lt = (row < col).astype(jnp.float32)
    cum_excl = jnp.dot(counts_row, lt, preferred_element_type=jnp.float32,
                       precision=_HI)
    upper = cum_excl + counts_row
    s = src_f[None, :]
    return ((s >= cum_excl.reshape(_NB, 1)) & (s < upper.reshape(_NB, 1))
            ).astype(jnp.float32)


def _prep1_body(x_ref, ws_ref, wd_ref, b1_ref, batch_ref, xs_ref, xd_ref, cnt_ref):
    x = x_ref[...]
    xs_ref[...] = jnp.dot(x, ws_ref[...], preferred_element_type=jnp.float32) + b1_ref[...]
    xd_ref[...] = jnp.dot(x, wd_ref[...], preferred_element_type=jnp.float32)
    oh = _onehot_from_ids(batch_ref[0, 0, :])
    cnt = jnp.sum(oh, axis=0)

    @pl.when(pl.program_id(0) == 0)
    def _():
        cnt_ref[...] = jnp.zeros_like(cnt_ref)

    cnt_ref[...] += jnp.concatenate(
        [cnt[None, :], jnp.zeros((7, _NB), jnp.float32)], axis=0)


def _prep1(x, ws, wd, b1, batch_r, interpret=False):
    n, d = x.shape
    g = n // _BN
    return pl.pallas_call(
        _prep1_body,
        grid=(g,),
        in_specs=[
            pl.BlockSpec((_BN, d), lambda i: (i, 0)),
            pl.BlockSpec((d, 64), lambda i: (0, 0)),
            pl.BlockSpec((d, 64), lambda i: (0, 0)),
            pl.BlockSpec((1, 64), lambda i: (0, 0)),
            pl.BlockSpec((1, 1, _BN), lambda i: (i, 0, 0)),
        ],
        out_specs=[
            pl.BlockSpec((_BN, 64), lambda i: (i, 0)),
            pl.BlockSpec((_BN, 64), lambda i: (i, 0)),
            pl.BlockSpec((8, _NB), lambda i: (0, 0)),
        ],
        out_shape=[
            jax.ShapeDtypeStruct((n, 64), jnp.float32),
            jax.ShapeDtypeStruct((n, 64), jnp.float32),
            jax.ShapeDtypeStruct((8, _NB), jnp.float32),
        ],
        interpret=interpret,
    )(x, ws, wd, b1, batch_r)


def _prep2_body(x_ref, ws_ref, wd_ref, b1_ref, batch_ref, uwe_ref, xs_ref, xd_ref):
    x = x_ref[...]
    oh = _onehot_from_ids(batch_ref[0, 0, :])
    xs_ref[...] = (jnp.dot(x, ws_ref[...], preferred_element_type=jnp.float32)
                   + jnp.dot(oh, uwe_ref[...], preferred_element_type=jnp.float32,
                             precision=_HI)
                   + b1_ref[...])
    xd_ref[...] = jnp.dot(x, wd_ref[...], preferred_element_type=jnp.float32)


def _prep2(x, ws, wd, b1, batch_r, uwe, interpret=False):
    n, d = x.shape
    g = n // _BN
    return pl.pallas_call(
        _prep2_body,
        grid=(g,),
        in_specs=[
            pl.BlockSpec((_BN, d), lambda i: (i, 0)),
            pl.BlockSpec((d, 64), lambda i: (0, 0)),
            pl.BlockSpec((d, 64), lambda i: (0, 0)),
            pl.BlockSpec((1, 64), lambda i: (0, 0)),
            pl.BlockSpec((1, 1, _BN), lambda i: (i, 0, 0)),
            pl.BlockSpec((_NB, 64), lambda i: (0, 0)),
        ],
        out_specs=[
            pl.BlockSpec((_BN, 64), lambda i: (i, 0)),
            pl.BlockSpec((_BN, 64), lambda i: (i, 0)),
        ],
        out_shape=[
            jax.ShapeDtypeStruct((n, 64), jnp.float32),
            jax.ShapeDtypeStruct((n, 64), jnp.float32),
        ],
        interpret=interpret,
    )(x, ws, wd, b1, batch_r, uwe)


def _edge1_body(xsg_ref, xdg_ref, eae_ref, eao_ref, srce_ref, srco_ref,
                cnt_ref, w1e_ref, w2b_ref, b2b_ref, eo_ref, ge_ref):
    cnt = cnt_ref[0:1, :]
    oht_e = _onehot_t_from_src(srce_ref[0, 0, :].astype(jnp.float32), cnt)
    oht_o = _onehot_t_from_src(srco_ref[0, 0, :].astype(jnp.float32), cnt)
    we = lax.dot_general(eae_ref[...], w1e_ref[...], (((0,), (0,)), ((), ())),
                         preferred_element_type=jnp.float32)
    wo = lax.dot_general(eao_ref[...], w1e_ref[...], (((0,), (0,)), ((), ())),
                         preferred_element_type=jnp.float32)
    h = xsg_ref[...] + xdg_ref[...] + jnp.concatenate([we, wo], axis=1)
    h = jnp.maximum(h, 0.0)
    ea = jnp.dot(h, w2b_ref[...], preferred_element_type=jnp.float32) + b2b_ref[...]
    eo_ref[...] = ea
    gs = (jnp.dot(oht_e, ea[:, 0:16], preferred_element_type=jnp.float32,
                  precision=_HI)
          + jnp.dot(oht_o, ea[:, 16:32], preferred_element_type=jnp.float32,
                    precision=_HI))
    ecnt = jnp.sum(oht_e, axis=1) + jnp.sum(oht_o, axis=1)
    upd = jnp.concatenate(
        [gs, ecnt[None, :], jnp.zeros((7, _NB), jnp.float32)], axis=0)

    @pl.when(pl.program_id(0) == 0)
    def _():
        ge_ref[...] = jnp.zeros_like(ge_ref)

    ge_ref[...] += upd


def _edge1(xsg, xdg, ea_e, ea_o, src_er, src_or, cnt, w1e, w2b, b2b,
           interpret=False):
    eh = xsg.shape[0]
    ed = ea_e.shape[0]
    beh = _BE // 2
    g = eh // beh
    return pl.pallas_call(
        _edge1_body,
        grid=(g,),
        in_specs=[
            pl.BlockSpec((beh, 128), lambda i: (i, 0)),
            pl.BlockSpec((beh, 128), lambda i: (i, 0)),
            pl.BlockSpec((ed, beh), lambda i: (0, i)),
            pl.BlockSpec((ed, beh), lambda i: (0, i)),
            pl.BlockSpec((1, 1, beh), lambda i: (i, 0, 0)),
            pl.BlockSpec((1, 1, beh), lambda i: (i, 0, 0)),
            pl.BlockSpec((8, _NB), lambda i: (0, 0)),
            pl.BlockSpec((ed, 64), lambda i: (0, 0)),
            pl.BlockSpec((128, 32), lambda i: (0, 0)),
            pl.BlockSpec((1, 32), lambda i: (0, 0)),
        ],
        out_specs=[
            pl.BlockSpec((beh, 32), lambda i: (i, 0)),
            pl.BlockSpec((24, _NB), lambda i: (0, 0)),
        ],
        out_shape=[
            jax.ShapeDtypeStruct((eh, 32), jnp.float32),
            jax.ShapeDtypeStruct((24, _NB), jnp.float32),
        ],
        interpret=interpret,
    )(xsg, xdg, ea_e, ea_o, src_er, src_or, cnt, w1e, w2b, b2b)


def _edge2_body(xsg_ref, xdg_ref, ea_ref, w1eb_ref, w2b_ref, b2b_ref, eo_ref):
    h = xsg_ref[...] + xdg_ref[...] + jnp.dot(
        ea_ref[...], w1eb_ref[...], preferred_element_type=jnp.float32)
    h = jnp.maximum(h, 0.0)
    eo_ref[...] = jnp.dot(h, w2b_ref[...], preferred_element_type=jnp.float32) + b2b_ref[...]


def _edge2(xsg, xdg, ea_in2, w1eb, w2b, b2b, interpret=False):
    eh = xsg.shape[0]
    beh = _BE // 2
    g = eh // beh
    return pl.pallas_call(
        _edge2_body,
        grid=(g,),
        in_specs=[
            pl.BlockSpec((beh, 128), lambda i: (i, 0)),
            pl.BlockSpec((beh, 128), lambda i: (i, 0)),
            pl.BlockSpec((beh, 32), lambda i: (i, 0)),
            pl.BlockSpec((32, 128), lambda i: (0, 0)),
            pl.BlockSpec((128, 32), lambda i: (0, 0)),
            pl.BlockSpec((1, 32), lambda i: (0, 0)),
        ],
        out_specs=[pl.BlockSpec((beh, 32), lambda i: (i, 0))],
        out_shape=[jax.ShapeDtypeStruct((eh, 32), jnp.float32)],
        interpret=interpret,
    )(xsg, xdg, ea_in2, w1eb, w2b, b2b)


def _node1_body(x_ref, p0_ref, p1_ref, degp_ref, batch_ref, w1x_ref,
                w1a_ref, b1_ref, w2_ref, b2_ref, xo_ref, gx_ref):
    deg = jnp.sum(degp_ref[0], axis=0)[:, None]
    agg = (p0_ref[...] + p1_ref[...]) / jnp.maximum(deg, 1.0)
    oh = _onehot_from_ids(batch_ref[0, 0, :])
    h = (jnp.dot(x_ref[...], w1x_ref[...], preferred_element_type=jnp.float32)
         + jnp.dot(agg, w1a_ref[...], preferred_element_type=jnp.float32)
         + b1_ref[...])
    h = jnp.maximum(h, 0.0)
    xo = jnp.dot(h, w2_ref[...], preferred_element_type=jnp.float32) + b2_ref[...]
    xo_ref[...] = xo
    gs = lax.dot_general(oh, xo, (((0,), (0,)), ((), ())),
                         preferred_element_type=jnp.float32, precision=_HI)

    @pl.when(pl.program_id(0) == 0)
    def _():
        gx_ref[...] = jnp.zeros_like(gx_ref)

    gx_ref[...] += gs


def _node1(x, p0, p1, degp, batch_r, w1x, w1a, b1, w2, b2, interpret=False):
    n, d = x.shape
    g = n // _BN
    return pl.pallas_call(
        _node1_body,
        grid=(g,),
        in_specs=[
            pl.BlockSpec((_BN, d), lambda i: (i, 0)),
            pl.BlockSpec((_BN, 16), lambda i: (i, 0)),
            pl.BlockSpec((_BN, 16), lambda i: (i, 0)),
            pl.BlockSpec((1, _NW, _BN), lambda i: (i, 0, 0)),
            pl.BlockSpec((1, 1, _BN), lambda i: (i, 0, 0)),
            pl.BlockSpec((d, 64), lambda i: (0, 0)),
            pl.BlockSpec((16, 64), lambda i: (0, 0)),
            pl.BlockSpec((1, 64), lambda i: (0, 0)),
            pl.BlockSpec((64, d), lambda i: (0, 0)),
            pl.BlockSpec((1, d), lambda i: (0, 0)),
        ],
        out_specs=[
            pl.BlockSpec((_BN, d), lambda i: (i, 0)),
            pl.BlockSpec((_NB, d), lambda i: (0, 0)),
        ],
        out_shape=[
            jax.ShapeDtypeStruct((n, d), jnp.float32),
            jax.ShapeDtypeStruct((_NB, d), jnp.float32),
        ],
        interpret=interpret,
    )(x, p0, p1, degp, batch_r, w1x, w1a, b1, w2, b2)


def _node2_body(x_ref, p0_ref, p1_ref, degp_ref, batch_ref, w1x_ref,
                w1a_ref, uwn_ref, b1_ref, w2_ref, b2_ref, gx_ref):
    deg = jnp.sum(degp_ref[0], axis=0)[:, None]
    agg = (p0_ref[...] + p1_ref[...]) / jnp.maximum(deg, 1.0)
    oh = _onehot_from_ids(batch_ref[0, 0, :])
    h = (jnp.dot(x_ref[...], w1x_ref[...], preferred_element_type=jnp.float32)
         + jnp.dot(agg, w1a_ref[...], preferred_element_type=jnp.float32)
         + jnp.dot(oh, uwn_ref[...], preferred_element_type=jnp.float32,
                   precision=_HI)
         + b1_ref[...])
    h = jnp.maximum(h, 0.0)
    xo = jnp.dot(h, w2_ref[...], preferred_element_type=jnp.float32) + b2_ref[...]
    gs = lax.dot_general(oh, xo, (((0,), (0,)), ((), ())),
                         preferred_element_type=jnp.float32, precision=_HI)

    @pl.when(pl.program_id(0) == 0)
    def _():
        gx_ref[...] = jnp.zeros_like(gx_ref)

    gx_ref[...] += gs


def _node2(x, p0, p1, degp, batch_r, w1x, w1a, uwn, b1, w2, b2, interpret=False):
    n, d = x.shape
    g = n // _BN
    return pl.pallas_call(
        _node2_body,
        grid=(g,),
        in_specs=[
            pl.BlockSpec((_BN, d), lambda i: (i, 0)),
            pl.BlockSpec((_BN, 16), lambda i: (i, 0)),
            pl.BlockSpec((_BN, 16), lambda i: (i, 0)),
            pl.BlockSpec((1, _NW, _BN), lambda i: (i, 0, 0)),
            pl.BlockSpec((1, 1, _BN), lambda i: (i, 0, 0)),
            pl.BlockSpec((d, 64), lambda i: (0, 0)),
            pl.BlockSpec((16, 64), lambda i: (0, 0)),
            pl.BlockSpec((_NB, 64), lambda i: (0, 0)),
            pl.BlockSpec((1, 64), lambda i: (0, 0)),
            pl.BlockSpec((64, d), lambda i: (0, 0)),
            pl.BlockSpec((1, d), lambda i: (0, 0)),
        ],
        out_specs=[pl.BlockSpec((_NB, d), lambda i: (0, 0))],
        out_shape=[jax.ShapeDtypeStruct((_NB, d), jnp.float32)],
        interpret=interpret,
    )(x, p0, p1, degp, batch_r, w1x, w1a, uwn, b1, w2, b2)


def _glob1_body(cnt_ref, ge_ref, gx_ref, wg_gx_ref, wg_ge_ref, b1_ref, w2_ref,
                b2_ref, we_ref, wn_ref, uwe_ref, uwn_ref):
    ncnt = cnt_ref[0:1, :]
    gx = gx_ref[...] / jnp.maximum(ncnt, 1.0).reshape(_NB, 1)
    ecnt = ge_ref[16:17, :]
    ge = ge_ref[0:16, :] / jnp.maximum(ecnt, 1.0).reshape(_NB, 1)
    h = (jnp.dot(gx, wg_gx_ref[...], preferred_element_type=jnp.float32)
         + jnp.dot(ge, wg_ge_ref[...], preferred_element_type=jnp.float32)
         + b1_ref[...])
    h = jnp.maximum(h, 0.0)
    u1 = jnp.dot(h, w2_ref[...], preferred_element_type=jnp.float32) + b2_ref[...]
    uwe_ref[...] = jnp.dot(u1, we_ref[...], preferred_element_type=jnp.float32)
    uwn_ref[...] = jnp.dot(u1, wn_ref[...], preferred_element_type=jnp.float32)


def _glob1(cnt, ge, gx, wg_gx, wg_ge, b1, w2, b2, we, wn, interpret=False):
    return pl.pallas_call(
        _glob1_body,
        out_shape=[
            jax.ShapeDtypeStruct((_NB, 64), jnp.float32),
            jax.ShapeDtypeStruct((_NB, 64), jnp.float32),
        ],
        interpret=interpret,
    )(cnt, ge, gx, wg_gx, wg_ge, b1, w2, b2, we, wn)


def _bn16(h, g, b):
    m = jnp.mean(h, axis=0, keepdims=True)
    v = jnp.mean((h - m) ** 2, axis=0, keepdims=True)
    return g * (h - m) / jnp.sqrt(v + 1e-5) + b


def _head_body(gx_ref, cnt_ref, aw1_ref, ab1_ref, ag1_ref, abe1_ref, aw2_ref,
               ab2_ref, ow1_ref, ob1_ref, og1_ref, obe1_ref, ow2_ref, ob2_ref,
               og2_ref, obe2_ref, ow3_ref, ob3_ref, act_ref, obj_ref):
    maxn = jnp.max(cnt_ref[0:1, :])
    outputs = gx_ref[...] / maxn
    h = _bn16(jnp.dot(outputs, aw1_ref[...], preferred_element_type=jnp.float32)
              + ab1_ref[...], ag1_ref[...], abe1_ref[...])
    act_ref[...] = jnp.dot(jnp.maximum(h, 0.0), aw2_ref[...],
                           preferred_element_type=jnp.float32) + ab2_ref[...]
    h = jnp.maximum(_bn16(
        jnp.dot(outputs, ow1_ref[...], preferred_element_type=jnp.float32)
        + ob1_ref[...], og1_ref[...], obe1_ref[...]), 0.0)
    h = jnp.maximum(_bn16(
        jnp.dot(h, ow2_ref[...], preferred_element_type=jnp.float32)
        + ob2_ref[...], og2_ref[...], obe2_ref[...]), 0.0)
    obj_ref[...] = jnp.dot(h, ow3_ref[...],
                           preferred_element_type=jnp.float32) + ob3_ref[...]


def _head(gx2, cnt, pa, po, interpret=False):
    args = (gx2, cnt,
            pa["W1"], pa["b1"].reshape(1, -1), pa["g1"].reshape(1, -1),
            pa["be1"].reshape(1, -1), pa["W2"], pa["b2"].reshape(1, -1),
            po["W1"], po["b1"].reshape(1, -1), po["g1"].reshape(1, -1),
            po["be1"].reshape(1, -1), po["W2"], po["b2"].reshape(1, -1),
            po["g2"].reshape(1, -1), po["be2"].reshape(1, -1), po["W3"],
            po["b3"].reshape(1, -1))
    return pl.pallas_call(
        _head_body,
        out_shape=[
            jax.ShapeDtypeStruct((_NB, 32), jnp.float32),
            jax.ShapeDtypeStruct((_NB, 64), jnp.float32),
        ],
        interpret=interpret,
    )(*args)


@functools.lru_cache(maxsize=None)
def _build_sc_gather(n, e):
    mesh = plsc.VectorSubcoreMesh(core_axis_name="c", subcore_axis_name="s")
    epw = e // _NW
    nrow = epw // _W
    ng = nrow // _GB
    grp = _GB * _W

    @functools.partial(
        pl.kernel, mesh=mesh,
        compiler_params=pltpu.CompilerParams(use_tc_tiling_on_sc=False),
        out_type=(jax.ShapeDtypeStruct((e, 64), jnp.float32),
                  jax.ShapeDtypeStruct((e, 64), jnp.float32)),
        scratch_types=[
            pltpu.VMEM((nrow, _W), jnp.int32),
            pltpu.VMEM((nrow, _W), jnp.int32),
            pltpu.VMEM((grp, 64), jnp.float32),
            pltpu.SemaphoreType.DMA,
        ],
    )
    def gk(xs_hbm, xd_hbm, src_hbm, dst_hbm, xsg_hbm, xdg_hbm,
           idxs, idxd, rows, sem):
        wid = lax.axis_index("s") * 2 + lax.axis_index("c")
        tb = wid * nrow
        base = wid * epw
        pltpu.sync_copy(src_hbm.at[pl.ds(tb, nrow)], idxs)
        pltpu.sync_copy(dst_hbm.at[pl.ds(tb, nrow)], idxd)

        def group(g, carry):
            off = base + g * grp
            cps = [pltpu.async_copy(xs_hbm.at[idxs.at[g * _GB + b]],
                                    rows.at[pl.ds(b * _W, _W)], sem)
                   for b in range(_GB)]
            for cp in cps:
                cp.wait()
            pltpu.sync_copy(rows, xsg_hbm.at[pl.ds(off, grp)])
            cps = [pltpu.async_copy(xd_hbm.at[idxd.at[g * _GB + b]],
                                    rows.at[pl.ds(b * _W, _W)], sem)
                   for b in range(_GB)]
            for cp in cps:
                cp.wait()
            pltpu.sync_copy(rows, xdg_hbm.at[pl.ds(off, grp)])
            return carry

        lax.fori_loop(0, ng, group, 0)

    return gk


@functools.lru_cache(maxsize=None)
def _build_sc_scatter(n, e, with_deg):
    mesh = plsc.VectorSubcoreMesh(core_axis_name="c", subcore_axis_name="s")
    epw = e // _NW
    nrow = epw // _W
    ng = nrow // _GB
    grp = _GB * _W
    nzw = 10
    rps = n // nzw
    nv = n // 16
    ev = epw // 16

    outs = [jax.ShapeDtypeStruct((2, n, 16), jnp.float32)]
    scratch = [
        pltpu.VMEM((nrow, _W), jnp.int32),
        pltpu.VMEM((grp, 16), jnp.float32),
        pltpu.VMEM_SHARED((n, 16), jnp.float32),
    ]
    if with_deg:
        outs.append(jax.ShapeDtypeStruct((_NW * n,), jnp.float32))
        scratch += [
            pltpu.VMEM((epw,), jnp.int32),
            pltpu.VMEM((n,), jnp.float32),
        ]

    def body(ea_hbm, dst_hbm, dstf_hbm, zeros_hbm, agg_hbm, deg_hbm,
             idxd, rows, accum, dflat, dacc):
        cid = lax.axis_index("c")
        sid = lax.axis_index("s")
        wid = sid * 2 + cid

        @pl.when(sid < nzw)
        def _():
            pltpu.sync_copy(zeros_hbm, accum.at[pl.ds(sid * rps, rps)])

        plsc.subcore_barrier()
        tb = wid * nrow
        base = wid * epw
        pltpu.sync_copy(dst_hbm.at[pl.ds(tb, nrow)], idxd)

        if with_deg:
            pltpu.sync_copy(dstf_hbm.at[pl.ds(base, epw)], dflat)
            zv = jnp.zeros((16,), jnp.float32)

            def zloop(j, c):
                dacc[pl.ds(j * 16, 16)] = zv
                return c

            lax.fori_loop(0, nv, zloop, 0)
            ones = jnp.full((16,), 1.0, jnp.float32)

            def dloop(j, c):
                idx = dflat[pl.ds(j * 16, 16)]
                plsc.addupdate_scatter(dacc, [idx], ones)
                return c

            lax.fori_loop(0, ev, dloop, 0)
            for blk in range(n // _BN):
                pltpu.sync_copy(
                    dacc.at[pl.ds(blk * _BN, _BN)],
                    deg_hbm.at[pl.ds(blk * _NW * _BN + wid * _BN, _BN)])

        def group(g, carry):
            pltpu.sync_copy(ea_hbm.at[pl.ds(base + g * grp, grp)], rows)
            for b in range(_GB):
                pltpu.sync_copy(rows.at[pl.ds(b * _W, _W)],
                                accum.at[idxd.at[g * _GB + b]], add=True)
            return carry

        lax.fori_loop(0, ng, group, 0)
        plsc.subcore_barrier()

        @pl.when(sid < nzw)
        def _():
            pltpu.sync_copy(accum.at[pl.ds(sid * rps, rps)],
                            agg_hbm.at[cid, pl.ds(sid * rps, rps)])

    if with_deg:
        def sk(ea_hbm, dst_hbm, dstf_hbm, zeros_hbm, agg_hbm, deg_hbm,
               idxd, rows, accum, dflat, dacc):
            body(ea_hbm, dst_hbm, dstf_hbm, zeros_hbm, agg_hbm, deg_hbm,
                 idxd, rows, accum, dflat, dacc)
    else:
        def sk(ea_hbm, dst_hbm, zeros_hbm, agg_hbm, idxd, rows, accum):
            body(ea_hbm, dst_hbm, None, zeros_hbm, agg_hbm, None,
                 idxd, rows, accum, None, None)

    return functools.partial(
        pl.kernel, mesh=mesh, out_type=tuple(outs),
        compiler_params=pltpu.CompilerParams(use_tc_tiling_on_sc=False,
                                             needs_layout_passes=False),
        scratch_types=scratch)(sk)


def kernel(x, edge_index, edge_attr, batch, params):
    n, d = x.shape
    e = edge_index.shape[1]
    src = edge_index[0].astype(jnp.int32)
    dst = edge_index[1].astype(jnp.int32)
    src2d = src.reshape(e // _W, _W)
    dst2d = dst.reshape(e // _W, _W)
    beh = _BE // 2
    srcp = src.reshape(e // 2, 2)
    src_er = srcp[:, 0].reshape(e // _BE, 1, beh)
    src_or = srcp[:, 1].reshape(e // _BE, 1, beh)
    eat = edge_attr.T.reshape(edge_attr.shape[1], e // 2, 2)
    ea_e = eat[:, :, 0]
    ea_o = eat[:, :, 1]
    batch_r = batch.astype(jnp.int32).reshape(n // _BN, 1, _BN)
    zeros_np = jnp.zeros((n // 10, 16), jnp.float32)

    p1, p2 = params["gnn1"], params["gnn2"]
    pe1, pn1, pg1 = p1["edge"], p1["node"], p1["glob"]
    pe2, pn2 = p2["edge"], p2["node"]

    gather = _build_sc_gather(n, e)
    scatter1 = _build_sc_scatter(n, e, True)
    scatter2 = _build_sc_scatter(n, e, False)

    xs1, xd1, cnt = _prep1(x, pe1["W1"][:d], pe1["W1"][d:2 * d],
                           pe1["b1"].reshape(1, -1), batch_r)
    xsg1, xdg1 = gather(xs1, xd1, src2d, dst2d)
    xsg1 = xsg1.reshape(e // 2, 128)
    xdg1 = xdg1.reshape(e // 2, 128)
    w2b1 = jax.scipy.linalg.block_diag(pe1["W2"], pe1["W2"])
    b2b1 = jnp.tile(pe1["b2"].reshape(1, -1), (1, 2))
    ea1, ge = _edge1(xsg1, xdg1, ea_e, ea_o, src_er, src_or, cnt,
                     pe1["W1"][2 * d:2 * d + 16], w2b1, b2b1)
    aggp, degf = scatter1(ea1.reshape(e, 16), dst2d, dst, zeros_np)
    degp = degf.reshape(n // _BN, _NW, _BN)
    x1, gx1 = _node1(x, aggp[0], aggp[1], degp, batch_r,
                     pn1["W1"][:d], pn1["W1"][d:d + 16],
                     pn1["b1"].reshape(1, -1), pn1["W2"],
                     pn1["b2"].reshape(1, -1))
    uwe2, uwn2 = _glob1(cnt, ge, gx1, pg1["W1"][16:16 + d],
                        pg1["W1"][16 + d:], pg1["b1"].reshape(1, -1),
                        pg1["W2"], pg1["b2"].reshape(1, -1),
                        pe2["W1"][2 * d + 16:], pn2["W1"][d + 16:])

    xs2, xd2 = _prep2(x1, pe2["W1"][:d], pe2["W1"][d:2 * d],
                      pe2["b1"].reshape(1, -1), batch_r, uwe2)
    xsg2, xdg2 = gather(xs2, xd2, src2d, dst2d)
    xsg2 = xsg2.reshape(e // 2, 128)
    xdg2 = xdg2.reshape(e // 2, 128)
    w1eb2 = jax.scipy.linalg.block_diag(pe2["W1"][2 * d:2 * d + 16],
                                        pe2["W1"][2 * d:2 * d + 16])
    w2b2 = jax.scipy.linalg.block_diag(pe2["W2"], pe2["W2"])
    b2b2 = jnp.tile(pe2["b2"].reshape(1, -1), (1, 2))
    (ea2,) = _edge2(xsg2, xdg2, ea1, w1eb2, w2b2, b2b2)
    res2 = scatter2(ea2.reshape(e, 16), dst2d, zeros_np)
    aggp2 = res2[0] if isinstance(res2, (tuple, list)) else res2
    (gx2,) = _node2(x1, aggp2[0], aggp2[1], degp, batch_r,
                    pn2["W1"][:d], pn2["W1"][d:d + 16], uwn2,
                    pn2["b1"].reshape(1, -1), pn2["W2"],
                    pn2["b2"].reshape(1, -1))
    act, obj = _head(gx2, cnt, params["action"], params["object"])
    return act, obj


# speedup vs baseline: 9.9143x; 1.3800x over previous
import functools

import jax
import jax.numpy as jnp
from jax import lax
from jax.experimental import pallas as pl
from jax.experimental.pallas import tpu as pltpu
from jax.experimental.pallas import tpu_sc as plsc

_BN = 1000
_BE = 6400
_W = 125
_GB = 8
_NB = 16
_NW = 32

_HI = lax.Precision.HIGHEST


def _iota16():
    return lax.broadcasted_iota(jnp.int32, (1, _NB), 1)


def _onehot_from_ids(ids):
    return (ids[:, None] == _iota16()).astype(jnp.float32)


def _onehot_t_from_src(src_f, counts_row):
    row = lax.broadcasted_iota(jnp.int32, (_NB, _NB), 0)
    col = lax.broadcasted_iota(jnp.int32, (_NB, _NB), 1)
    lt = (row < col).astype(jnp.float32)
    cum_excl = jnp.dot(counts_row, lt, preferred_element_type=jnp.float32,
                       precision=_HI)
    upper = cum_excl + counts_row
    s = src_f[None, :]
    return ((s >= cum_excl.reshape(_NB, 1)) & (s < upper.reshape(_NB, 1))
            ).astype(jnp.float32)


def _prep1_body(x_ref, ws_ref, wd_ref, b1_ref, batch_ref, xs_ref, xd_ref, cnt_ref):
    x = x_ref[...]
    xs_ref[...] = jnp.dot(x, ws_ref[...], preferred_element_type=jnp.float32) + b1_ref[...]
    xd_ref[...] = jnp.dot(x, wd_ref[...], preferred_element_type=jnp.float32)
    oh = _onehot_from_ids(batch_ref[0, 0, :])
    cnt = jnp.sum(oh, axis=0)

    @pl.when(pl.program_id(0) == 0)
    def _():
        cnt_ref[...] = jnp.zeros_like(cnt_ref)

    cnt_ref[...] += jnp.concatenate(
        [cnt[None, :], jnp.zeros((7, _NB), jnp.float32)], axis=0)


def _prep1(x, ws, wd, b1, batch_r, interpret=False):
    n, d = x.shape
    g = n // _BN
    return pl.pallas_call(
        _prep1_body,
        grid=(g,),
        in_specs=[
            pl.BlockSpec((_BN, d), lambda i: (i, 0)),
            pl.BlockSpec((d, 64), lambda i: (0, 0)),
            pl.BlockSpec((d, 64), lambda i: (0, 0)),
            pl.BlockSpec((1, 64), lambda i: (0, 0)),
            pl.BlockSpec((1, 1, _BN), lambda i: (i, 0, 0)),
        ],
        out_specs=[
            pl.BlockSpec((_BN, 64), lambda i: (i, 0)),
            pl.BlockSpec((_BN, 64), lambda i: (i, 0)),
            pl.BlockSpec((8, _NB), lambda i: (0, 0)),
        ],
        out_shape=[
            jax.ShapeDtypeStruct((n, 64), jnp.float32),
            jax.ShapeDtypeStruct((n, 64), jnp.float32),
            jax.ShapeDtypeStruct((8, _NB), jnp.float32),
        ],
        interpret=interpret,
    )(x, ws, wd, b1, batch_r)


def _prep2_body(x_ref, ws_ref, wd_ref, b1_ref, batch_ref, uwe_ref, xs_ref, xd_ref):
    x = x_ref[...]
    oh = _onehot_from_ids(batch_ref[0, 0, :])
    xs_ref[...] = (jnp.dot(x, ws_ref[...], preferred_element_type=jnp.float32)
                   + jnp.dot(oh, uwe_ref[...], preferred_element_type=jnp.float32,
                             precision=_HI)
                   + b1_ref[...])
    xd_ref[...] = jnp.dot(x, wd_ref[...], preferred_element_type=jnp.float32)


def _prep2(x, ws, wd, b1, batch_r, uwe, interpret=False):
    n, d = x.shape
    g = n // _BN
    return pl.pallas_call(
        _prep2_body,
        grid=(g,),
        in_specs=[
            pl.BlockSpec((_BN, d), lambda i: (i, 0)),
            pl.BlockSpec((d, 64), lambda i: (0, 0)),
            pl.BlockSpec((d, 64), lambda i: (0, 0)),
            pl.BlockSpec((1, 64), lambda i: (0, 0)),
            pl.BlockSpec((1, 1, _BN), lambda i: (i, 0, 0)),
            pl.BlockSpec((_NB, 64), lambda i: (0, 0)),
        ],
        out_specs=[
            pl.BlockSpec((_BN, 64), lambda i: (i, 0)),
            pl.BlockSpec((_BN, 64), lambda i: (i, 0)),
        ],
        out_shape=[
            jax.ShapeDtypeStruct((n, 64), jnp.float32),
            jax.ShapeDtypeStruct((n, 64), jnp.float32),
        ],
        interpret=interpret,
    )(x, ws, wd, b1, batch_r, uwe)


def _edge1_body(xsg_ref, xdg_ref, eap_ref, srce_ref, srco_ref,
                cnt_ref, w1eb_ref, w2b_ref, b2b_ref, eo_ref, ge_ref):
    cnt = cnt_ref[0:1, :]
    oht_e = _onehot_t_from_src(srce_ref[0, 0, :].astype(jnp.float32), cnt)
    oht_o = _onehot_t_from_src(srco_ref[0, 0, :].astype(jnp.float32), cnt)
    h = xsg_ref[...] + xdg_ref[...] + jnp.dot(
        eap_ref[...], w1eb_ref[...], preferred_element_type=jnp.float32)
    h = jnp.maximum(h, 0.0)
    ea = jnp.dot(h, w2b_ref[...], preferred_element_type=jnp.float32) + b2b_ref[...]
    eo_ref[...] = ea
    gs = (jnp.dot(oht_e, ea[:, 0:16], preferred_element_type=jnp.float32,
                  precision=_HI)
          + jnp.dot(oht_o, ea[:, 16:32], preferred_element_type=jnp.float32,
                    precision=_HI))
    ecnt = jnp.sum(oht_e, axis=1) + jnp.sum(oht_o, axis=1)
    upd = jnp.concatenate(
        [gs, ecnt[None, :], jnp.zeros((7, _NB), jnp.float32)], axis=0)

    @pl.when(pl.program_id(0) == 0)
    def _():
        ge_ref[...] = jnp.zeros_like(ge_ref)

    ge_ref[...] += upd


def _edge1(xsg, xdg, eap, src_er, src_or, cnt, w1eb, w2b, b2b,
           interpret=False):
    eh = xsg.shape[0]
    beh = _BE // 2
    g = eh // beh
    return pl.pallas_call(
        _edge1_body,
        grid=(g,),
        in_specs=[
            pl.BlockSpec((beh, 128), lambda i: (i, 0)),
            pl.BlockSpec((beh, 128), lambda i: (i, 0)),
            pl.BlockSpec((beh, 32), lambda i: (i, 0)),
            pl.BlockSpec((1, 1, beh), lambda i: (i, 0, 0)),
            pl.BlockSpec((1, 1, beh), lambda i: (i, 0, 0)),
            pl.BlockSpec((8, _NB), lambda i: (0, 0)),
            pl.BlockSpec((32, 128), lambda i: (0, 0)),
            pl.BlockSpec((128, 32), lambda i: (0, 0)),
            pl.BlockSpec((1, 32), lambda i: (0, 0)),
        ],
        out_specs=[
            pl.BlockSpec((beh, 32), lambda i: (i, 0)),
            pl.BlockSpec((24, _NB), lambda i: (0, 0)),
        ],
        out_shape=[
            jax.ShapeDtypeStruct((eh, 32), jnp.float32),
            jax.ShapeDtypeStruct((24, _NB), jnp.float32),
        ],
        interpret=interpret,
    )(xsg, xdg, eap, src_er, src_or, cnt, w1eb, w2b, b2b)


def _edge2_body(xsg_ref, xdg_ref, ea_ref, w1eb_ref, w2b_ref, b2b_ref, eo_ref):
    h = xsg_ref[...] + xdg_ref[...] + jnp.dot(
        ea_ref[...], w1eb_ref[...], preferred_element_type=jnp.float32)
    h = jnp.maximum(h, 0.0)
    eo_ref[...] = jnp.dot(h, w2b_ref[...], preferred_element_type=jnp.float32) + b2b_ref[...]


def _edge2(xsg, xdg, ea_in2, w1eb, w2b, b2b, interpret=False):
    eh = xsg.shape[0]
    beh = _BE // 2
    g = eh // beh
    return pl.pallas_call(
        _edge2_body,
        grid=(g,),
        in_specs=[
            pl.BlockSpec((beh, 128), lambda i: (i, 0)),
            pl.BlockSpec((beh, 128), lambda i: (i, 0)),
            pl.BlockSpec((beh, 32), lambda i: (i, 0)),
            pl.BlockSpec((32, 128), lambda i: (0, 0)),
            pl.BlockSpec((128, 32), lambda i: (0, 0)),
            pl.BlockSpec((1, 32), lambda i: (0, 0)),
        ],
        out_specs=[pl.BlockSpec((beh, 32), lambda i: (i, 0))],
        out_shape=[jax.ShapeDtypeStruct((eh, 32), jnp.float32)],
        interpret=interpret,
    )(xsg, xdg, ea_in2, w1eb, w2b, b2b)


def _node1_body(x_ref, p0_ref, p1_ref, degp_ref, batch_ref, w1x_ref,
                w1a_ref, b1_ref, w2_ref, b2_ref, xo_ref, gx_ref):
    deg = jnp.sum(degp_ref[0], axis=0)[:, None]
    agg = (p0_ref[...] + p1_ref[...]) / jnp.maximum(deg, 1.0)
    oh = _onehot_from_ids(batch_ref[0, 0, :])
    h = (jnp.dot(x_ref[...], w1x_ref[...], preferred_element_type=jnp.float32)
         + jnp.dot(agg, w1a_ref[...], preferred_element_type=jnp.float32)
         + b1_ref[...])
    h = jnp.maximum(h, 0.0)
    xo = jnp.dot(h, w2_ref[...], preferred_element_type=jnp.float32) + b2_ref[...]
    xo_ref[...] = xo
    gs = lax.dot_general(oh, xo, (((0,), (0,)), ((), ())),
                         preferred_element_type=jnp.float32, precision=_HI)

    @pl.when(pl.program_id(0) == 0)
    def _():
        gx_ref[...] = jnp.zeros_like(gx_ref)

    gx_ref[...] += gs


def _node1(x, p0, p1, degp, batch_r, w1x, w1a, b1, w2, b2, interpret=False):
    n, d = x.shape
    g = n // _BN
    return pl.pallas_call(
        _node1_body,
        grid=(g,),
        in_specs=[
            pl.BlockSpec((_BN, d), lambda i: (i, 0)),
            pl.BlockSpec((_BN, 16), lambda i: (i, 0)),
            pl.BlockSpec((_BN, 16), lambda i: (i, 0)),
            pl.BlockSpec((1, _NW, _BN), lambda i: (i, 0, 0)),
            pl.BlockSpec((1, 1, _BN), lambda i: (i, 0, 0)),
            pl.BlockSpec((d, 64), lambda i: (0, 0)),
            pl.BlockSpec((16, 64), lambda i: (0, 0)),
            pl.BlockSpec((1, 64), lambda i: (0, 0)),
            pl.BlockSpec((64, d), lambda i: (0, 0)),
            pl.BlockSpec((1, d), lambda i: (0, 0)),
        ],
        out_specs=[
            pl.BlockSpec((_BN, d), lambda i: (i, 0)),
            pl.BlockSpec((_NB, d), lambda i: (0, 0)),
        ],
        out_shape=[
            jax.ShapeDtypeStruct((n, d), jnp.float32),
            jax.ShapeDtypeStruct((_NB, d), jnp.float32),
        ],
        interpret=interpret,
    )(x, p0, p1, degp, batch_r, w1x, w1a, b1, w2, b2)


def _node2_body(x_ref, p0_ref, p1_ref, degp_ref, batch_ref, w1x_ref,
                w1a_ref, uwn_ref, b1_ref, w2_ref, b2_ref, gx_ref):
    deg = jnp.sum(degp_ref[0], axis=0)[:, None]
    agg = (p0_ref[...] + p1_ref[...]) / jnp.maximum(deg, 1.0)
    oh = _onehot_from_ids(batch_ref[0, 0, :])
    h = (jnp.dot(x_ref[...], w1x_ref[...], preferred_element_type=jnp.float32)
         + jnp.dot(agg, w1a_ref[...], preferred_element_type=jnp.float32)
         + jnp.dot(oh, uwn_ref[...], preferred_element_type=jnp.float32,
                   precision=_HI)
         + b1_ref[...])
    h = jnp.maximum(h, 0.0)
    xo = jnp.dot(h, w2_ref[...], preferred_element_type=jnp.float32) + b2_ref[...]
    gs = lax.dot_general(oh, xo, (((0,), (0,)), ((), ())),
                         preferred_element_type=jnp.float32, precision=_HI)

    @pl.when(pl.program_id(0) == 0)
    def _():
        gx_ref[...] = jnp.zeros_like(gx_ref)

    gx_ref[...] += gs


def _node2(x, p0, p1, degp, batch_r, w1x, w1a, uwn, b1, w2, b2, interpret=False):
    n, d = x.shape
    g = n // _BN
    return pl.pallas_call(
        _node2_body,
        grid=(g,),
        in_specs=[
            pl.BlockSpec((_BN, d), lambda i: (i, 0)),
            pl.BlockSpec((_BN, 16), lambda i: (i, 0)),
            pl.BlockSpec((_BN, 16), lambda i: (i, 0)),
            pl.BlockSpec((1, _NW, _BN), lambda i: (i, 0, 0)),
            pl.BlockSpec((1, 1, _BN), lambda i: (i, 0, 0)),
            pl.BlockSpec((d, 64), lambda i: (0, 0)),
            pl.BlockSpec((16, 64), lambda i: (0, 0)),
            pl.BlockSpec((_NB, 64), lambda i: (0, 0)),
            pl.BlockSpec((1, 64), lambda i: (0, 0)),
            pl.BlockSpec((64, d), lambda i: (0, 0)),
            pl.BlockSpec((1, d), lambda i: (0, 0)),
        ],
        out_specs=[pl.BlockSpec((_NB, d), lambda i: (0, 0))],
        out_shape=[jax.ShapeDtypeStruct((_NB, d), jnp.float32)],
        interpret=interpret,
    )(x, p0, p1, degp, batch_r, w1x, w1a, uwn, b1, w2, b2)


def _glob1_body(cnt_ref, ge_ref, gx_ref, wg_gx_ref, wg_ge_ref, b1_ref, w2_ref,
                b2_ref, we_ref, wn_ref, uwe_ref, uwn_ref):
    ncnt = cnt_ref[0:1, :]
    gx = gx_ref[...] / jnp.maximum(ncnt, 1.0).reshape(_NB, 1)
    ecnt = ge_ref[16:17, :]
    ge = ge_ref[0:16, :] / jnp.maximum(ecnt, 1.0).reshape(_NB, 1)
    h = (jnp.dot(gx, wg_gx_ref[...], preferred_element_type=jnp.float32)
         + jnp.dot(ge, wg_ge_ref[...], preferred_element_type=jnp.float32)
         + b1_ref[...])
    h = jnp.maximum(h, 0.0)
    u1 = jnp.dot(h, w2_ref[...], preferred_element_type=jnp.float32) + b2_ref[...]
    uwe_ref[...] = jnp.dot(u1, we_ref[...], preferred_element_type=jnp.float32)
    uwn_ref[...] = jnp.dot(u1, wn_ref[...], preferred_element_type=jnp.float32)


def _glob1(cnt, ge, gx, wg_gx, wg_ge, b1, w2, b2, we, wn, interpret=False):
    return pl.pallas_call(
        _glob1_body,
        out_shape=[
            jax.ShapeDtypeStruct((_NB, 64), jnp.float32),
            jax.ShapeDtypeStruct((_NB, 64), jnp.float32),
        ],
        interpret=interpret,
    )(cnt, ge, gx, wg_gx, wg_ge, b1, w2, b2, we, wn)


def _bn16(h, g, b):
    m = jnp.mean(h, axis=0, keepdims=True)
    v = jnp.mean((h - m) ** 2, axis=0, keepdims=True)
    return g * (h - m) / jnp.sqrt(v + 1e-5) + b


def _head_body(gx_ref, cnt_ref, aw1_ref, ab1_ref, ag1_ref, abe1_ref, aw2_ref,
               ab2_ref, ow1_ref, ob1_ref, og1_ref, obe1_ref, ow2_ref, ob2_ref,
               og2_ref, obe2_ref, ow3_ref, ob3_ref, act_ref, obj_ref):
    maxn = jnp.max(cnt_ref[0:1, :])
    outputs = gx_ref[...] / maxn
    h = _bn16(jnp.dot(outputs, aw1_ref[...], preferred_element_type=jnp.float32)
              + ab1_ref[...], ag1_ref[...], abe1_ref[...])
    act_ref[...] = jnp.dot(jnp.maximum(h, 0.0), aw2_ref[...],
                           preferred_element_type=jnp.float32) + ab2_ref[...]
    h = jnp.maximum(_bn16(
        jnp.dot(outputs, ow1_ref[...], preferred_element_type=jnp.float32)
        + ob1_ref[...], og1_ref[...], obe1_ref[...]), 0.0)
    h = jnp.maximum(_bn16(
        jnp.dot(h, ow2_ref[...], preferred_element_type=jnp.float32)
        + ob2_ref[...], og2_ref[...], obe2_ref[...]), 0.0)
    obj_ref[...] = jnp.dot(h, ow3_ref[...],
                           preferred_element_type=jnp.float32) + ob3_ref[...]


def _head(gx2, cnt, pa, po, interpret=False):
    args = (gx2, cnt,
            pa["W1"], pa["b1"].reshape(1, -1), pa["g1"].reshape(1, -1),
            pa["be1"].reshape(1, -1), pa["W2"], pa["b2"].reshape(1, -1),
            po["W1"], po["b1"].reshape(1, -1), po["g1"].reshape(1, -1),
            po["be1"].reshape(1, -1), po["W2"], po["b2"].reshape(1, -1),
            po["g2"].reshape(1, -1), po["be2"].reshape(1, -1), po["W3"],
            po["b3"].reshape(1, -1))
    return pl.pallas_call(
        _head_body,
        out_shape=[
            jax.ShapeDtypeStruct((_NB, 32), jnp.float32),
            jax.ShapeDtypeStruct((_NB, 64), jnp.float32),
        ],
        interpret=interpret,
    )(*args)


@functools.lru_cache(maxsize=None)
def _build_sc_gather(n, e):
    mesh = plsc.VectorSubcoreMesh(core_axis_name="c", subcore_axis_name="s")
    epw = e // _NW
    nrow = epw // _W
    ng = nrow // _GB
    grp = _GB * _W

    @functools.partial(
        pl.kernel, mesh=mesh,
        compiler_params=pltpu.CompilerParams(use_tc_tiling_on_sc=False),
        out_type=(jax.ShapeDtypeStruct((e, 64), jnp.float32),
                  jax.ShapeDtypeStruct((e, 64), jnp.float32)),
        scratch_types=[
            pltpu.VMEM((nrow, _W), jnp.int32),
            pltpu.VMEM((nrow, _W), jnp.int32),
            pltpu.VMEM((grp, 64), jnp.float32),
            pltpu.SemaphoreType.DMA,
        ],
    )
    def gk(xs_hbm, xd_hbm, src_hbm, dst_hbm, xsg_hbm, xdg_hbm,
           idxs, idxd, rows, sem):
        wid = lax.axis_index("s") * 2 + lax.axis_index("c")
        tb = wid * nrow
        base = wid * epw
        pltpu.sync_copy(src_hbm.at[pl.ds(tb, nrow)], idxs)
        pltpu.sync_copy(dst_hbm.at[pl.ds(tb, nrow)], idxd)

        def group(g, carry):
            off = base + g * grp
            cps = [pltpu.async_copy(xs_hbm.at[idxs.at[g * _GB + b]],
                                    rows.at[pl.ds(b * _W, _W)], sem)
                   for b in range(_GB)]
            for cp in cps:
                cp.wait()
            pltpu.sync_copy(rows, xsg_hbm.at[pl.ds(off, grp)])
            cps = [pltpu.async_copy(xd_hbm.at[idxd.at[g * _GB + b]],
                                    rows.at[pl.ds(b * _W, _W)], sem)
                   for b in range(_GB)]
            for cp in cps:
                cp.wait()
            pltpu.sync_copy(rows, xdg_hbm.at[pl.ds(off, grp)])
            return carry

        lax.fori_loop(0, ng, group, 0)

    return gk


@functools.lru_cache(maxsize=None)
def _build_sc_scatter(n, e, with_deg):
    mesh = plsc.VectorSubcoreMesh(core_axis_name="c", subcore_axis_name="s")
    epw = e // _NW
    nrow = epw // _W
    ng = nrow // _GB
    grp = _GB * _W
    nzw = 10
    rps = n // nzw
    nv = n // 16
    ev = epw // 16

    outs = [jax.ShapeDtypeStruct((2, n, 16), jnp.float32)]
    scratch = [
        pltpu.VMEM((nrow, _W), jnp.int32),
        pltpu.VMEM((grp, 16), jnp.float32),
        pltpu.VMEM_SHARED((n, 16), jnp.float32),
    ]
    if with_deg:
        outs.append(jax.ShapeDtypeStruct((_NW * n,), jnp.float32))
        scratch += [
            pltpu.VMEM((epw,), jnp.int32),
            pltpu.VMEM((n,), jnp.float32),
        ]

    def body(ea_hbm, dst_hbm, dstf_hbm, zeros_hbm, agg_hbm, deg_hbm,
             idxd, rows, accum, dflat, dacc):
        cid = lax.axis_index("c")
        sid = lax.axis_index("s")
        wid = sid * 2 + cid

        @pl.when(sid < nzw)
        def _():
            pltpu.sync_copy(zeros_hbm, accum.at[pl.ds(sid * rps, rps)])

        plsc.subcore_barrier()
        tb = wid * nrow
        base = wid * epw
        pltpu.sync_copy(dst_hbm.at[pl.ds(tb, nrow)], idxd)

        if with_deg:
            pltpu.sync_copy(dstf_hbm.at[pl.ds(base, epw)], dflat)
            zv = jnp.zeros((16,), jnp.float32)

            def zloop(j, c):
                dacc[pl.ds(j * 16, 16)] = zv
                return c

            lax.fori_loop(0, nv, zloop, 0)
            ones = jnp.full((16,), 1.0, jnp.float32)

            def dloop(j, c):
                idx = dflat[pl.ds(j * 16, 16)]
                plsc.addupdate_scatter(dacc, [idx], ones)
                return c

            lax.fori_loop(0, ev, dloop, 0)
            for blk in range(n // _BN):
                pltpu.sync_copy(
                    dacc.at[pl.ds(blk * _BN, _BN)],
                    deg_hbm.at[pl.ds(blk * _NW * _BN + wid * _BN, _BN)])

        def group(g, carry):
            pltpu.sync_copy(ea_hbm.at[pl.ds(base + g * grp, grp)], rows)
            for b in range(_GB):
                pltpu.sync_copy(rows.at[pl.ds(b * _W, _W)],
                                accum.at[idxd.at[g * _GB + b]], add=True)
            return carry

        lax.fori_loop(0, ng, group, 0)
        plsc.subcore_barrier()

        @pl.when(sid < nzw)
        def _():
            pltpu.sync_copy(accum.at[pl.ds(sid * rps, rps)],
                            agg_hbm.at[cid, pl.ds(sid * rps, rps)])

    if with_deg:
        def sk(ea_hbm, dst_hbm, dstf_hbm, zeros_hbm, agg_hbm, deg_hbm,
               idxd, rows, accum, dflat, dacc):
            body(ea_hbm, dst_hbm, dstf_hbm, zeros_hbm, agg_hbm, deg_hbm,
                 idxd, rows, accum, dflat, dacc)
    else:
        def sk(ea_hbm, dst_hbm, zeros_hbm, agg_hbm, idxd, rows, accum):
            body(ea_hbm, dst_hbm, None, zeros_hbm, agg_hbm, None,
                 idxd, rows, accum, None, None)

    return functools.partial(
        pl.kernel, mesh=mesh, out_type=tuple(outs),
        compiler_params=pltpu.CompilerParams(use_tc_tiling_on_sc=False,
                                             needs_layout_passes=False),
        scratch_types=scratch)(sk)


def kernel(x, edge_index, edge_attr, batch, params):
    n, d = x.shape
    e = edge_index.shape[1]
    src = edge_index[0].astype(jnp.int32)
    dst = edge_index[1].astype(jnp.int32)
    src2d = src.reshape(e // _W, _W)
    dst2d = dst.reshape(e // _W, _W)
    beh = _BE // 2
    srcp = src.reshape(e // 2, 2)
    src_er = srcp[:, 0].reshape(e // _BE, 1, beh)
    src_or = srcp[:, 1].reshape(e // _BE, 1, beh)
    eap = edge_attr.reshape(e // 2, 2 * edge_attr.shape[1])
    batch_r = batch.astype(jnp.int32).reshape(n // _BN, 1, _BN)
    zeros_np = jnp.zeros((n // 10, 16), jnp.float32)

    p1, p2 = params["gnn1"], params["gnn2"]
    pe1, pn1, pg1 = p1["edge"], p1["node"], p1["glob"]
    pe2, pn2 = p2["edge"], p2["node"]

    gather = _build_sc_gather(n, e)
    scatter1 = _build_sc_scatter(n, e, True)
    scatter2 = _build_sc_scatter(n, e, False)

    xs1, xd1, cnt = _prep1(x, pe1["W1"][:d], pe1["W1"][d:2 * d],
                           pe1["b1"].reshape(1, -1), batch_r)
    xsg1, xdg1 = gather(xs1, xd1, src2d, dst2d)
    xsg1 = xsg1.reshape(e // 2, 128)
    xdg1 = xdg1.reshape(e // 2, 128)
    w1eb1 = jax.scipy.linalg.block_diag(pe1["W1"][2 * d:2 * d + 16],
                                        pe1["W1"][2 * d:2 * d + 16])
    w2b1 = jax.scipy.linalg.block_diag(pe1["W2"], pe1["W2"])
    b2b1 = jnp.tile(pe1["b2"].reshape(1, -1), (1, 2))
    ea1, ge = _edge1(xsg1, xdg1, eap, src_er, src_or, cnt,
                     w1eb1, w2b1, b2b1)
    aggp, degf = scatter1(ea1.reshape(e, 16), dst2d, dst, zeros_np)
    degp = degf.reshape(n // _BN, _NW, _BN)
    x1, gx1 = _node1(x, aggp[0], aggp[1], degp, batch_r,
                     pn1["W1"][:d], pn1["W1"][d:d + 16],
                     pn1["b1"].reshape(1, -1), pn1["W2"],
                     pn1["b2"].reshape(1, -1))
    uwe2, uwn2 = _glob1(cnt, ge, gx1, pg1["W1"][16:16 + d],
                        pg1["W1"][16 + d:], pg1["b1"].reshape(1, -1),
                        pg1["W2"], pg1["b2"].reshape(1, -1),
                        pe2["W1"][2 * d + 16:], pn2["W1"][d + 16:])

    xs2, xd2 = _prep2(x1, pe2["W1"][:d], pe2["W1"][d:2 * d],
                      pe2["b1"].reshape(1, -1), batch_r, uwe2)
    xsg2, xdg2 = gather(xs2, xd2, src2d, dst2d)
    xsg2 = xsg2.reshape(e // 2, 128)
    xdg2 = xdg2.reshape(e // 2, 128)
    w1eb2 = jax.scipy.linalg.block_diag(pe2["W1"][2 * d:2 * d + 16],
                                        pe2["W1"][2 * d:2 * d + 16])
    w2b2 = jax.scipy.linalg.block_diag(pe2["W2"], pe2["W2"])
    b2b2 = jnp.tile(pe2["b2"].reshape(1, -1), (1, 2))
    (ea2,) = _edge2(xsg2, xdg2, ea1, w1eb2, w2b2, b2b2)
    res2 = scatter2(ea2.reshape(e, 16), dst2d, zeros_np)
    aggp2 = res2[0] if isinstance(res2, (tuple, list)) else res2
    (gx2,) = _node2(x1, aggp2[0], aggp2[1], degp, batch_r,
                    pn2["W1"][:d], pn2["W1"][d:d + 16], uwn2,
                    pn2["b1"].reshape(1, -1), pn2["W2"],
                    pn2["b2"].reshape(1, -1))
    act, obj = _head(gx2, cnt, params["action"], params["object"])
    return act, obj
